# Initial kernel scaffold; baseline (speedup 1.0000x reference)
import jax
import jax.numpy as jnp
from jax.experimental import pallas as pl


def kernel(x, edge_index, W_embed, b_embed, W_sage0, b_sage0, bn_g0, bn_b0, bn_m0, bn_v0, W_sage1, b_sage1, bn_g1, bn_b1, bn_m1, bn_v1, W_sage2, b_sage2, bn_g2, bn_b2, bn_m2, bn_v2, W_mlp0, b_mlp0, W_mlp1, b_mlp1, W_mlp2, b_mlp2):
    raise NotImplementedError("write your pallas kernel here")


# speedup vs baseline: 2.8118x; 2.8118x over previous
import functools

import jax
import jax.numpy as jnp
from jax import lax
from jax.experimental import pallas as pl
from jax.experimental.pallas import tpu as pltpu
from jax.experimental.pallas import tpu_sc as plsc

N = 10000
E = 160000
DIN = 1024
H = 256
HH = 128
NSUB = 16
ES = E // NSUB
CW = 128
NCH = (ES + CW - 1) // CW
TAIL = ES - (NCH - 1) * CW
PADE = NCH * CW
NDUM = 8
NCOPY = 10
CPR = N // NCOPY
BN_EPS = 1e-5

_PREC = jax.lax.Precision.HIGHEST

_sc_mesh = plsc.VectorSubcoreMesh(core_axis_name="c", subcore_axis_name="s",
                                  num_cores=2, num_subcores=NSUB)


@functools.partial(
    pl.kernel,
    out_type=jax.ShapeDtypeStruct((N, HH), jnp.float32),
    mesh=_sc_mesh,
    scratch_types=[
        pltpu.VMEM((NCH, CW), jnp.int32),
        pltpu.VMEM((CW, HH), jnp.float32),
        pltpu.VMEM_SHARED((N + NDUM, HH), jnp.float32),
        pltpu.SemaphoreType.DMA,
    ],
)
def _deg_kernel(dstp, ones, zeros, deg_out, dst_v, ones_v, accum, sem):
    c = lax.axis_index("c")
    s = lax.axis_index("s")

    @pl.when(c == 0)
    def _():
        pl.when(s < NCOPY)(
            lambda: pltpu.sync_copy(zeros, accum.at[pl.ds(s * CPR, CPR)]))
        pltpu.sync_copy(dstp.at[s], dst_v)
        pltpu.sync_copy(ones, ones_v)
        plsc.subcore_barrier()

        def chunk(j, carry):
            pltpu.sync_copy(ones_v, accum.at[dst_v.at[j]], add=True)
            return carry

        lax.fori_loop(0, NCH, chunk, 0)
        plsc.subcore_barrier()
        sl = pl.ds(s * CPR, CPR)
        pl.when(s < NCOPY)(
            lambda: pltpu.sync_copy(accum.at[sl], deg_out.at[sl]))


@functools.partial(
    pl.kernel,
    out_type=(jax.ShapeDtypeStruct((N, HH), jnp.float32),
              jax.ShapeDtypeStruct((N, HH), jnp.float32)),
    mesh=_sc_mesh,
    scratch_types=[
        pltpu.VMEM((NCH, CW), jnp.int32),
        pltpu.VMEM((NCH, CW), jnp.int32),
        pltpu.VMEM((CW, HH), jnp.float32),
        pltpu.VMEM_SHARED((N + NDUM, HH), jnp.float32),
        pltpu.SemaphoreType.DMA,
    ],
)
def _seg_kernel(h_lo, h_hi, srcp, dstp, zeros, c_lo, c_hi,
                src_v, dst_v, rows_v, accum, sem):
    c = lax.axis_index("c")
    s = lax.axis_index("s")
    pl.when(s < NCOPY)(
        lambda: pltpu.sync_copy(zeros, accum.at[pl.ds(s * CPR, CPR)]))
    pltpu.sync_copy(srcp.at[s], src_v)
    pltpu.sync_copy(dstp.at[s], dst_v)
    plsc.subcore_barrier()

    def run(table):
        def chunk(j, carry):
            pltpu.async_copy(table.at[src_v.at[j]], rows_v, sem).wait()
            pltpu.sync_copy(rows_v, accum.at[dst_v.at[j]], add=True)
            return carry

        lax.fori_loop(0, NCH, chunk, 0)

    pl.when(c == 0)(lambda: run(h_lo))
    pl.when(c == 1)(lambda: run(h_hi))
    plsc.subcore_barrier()
    sl = pl.ds(s * CPR, CPR)

    @pl.when(s < NCOPY)
    def _():
        pl.when(c == 0)(lambda: pltpu.sync_copy(accum.at[sl], c_lo.at[sl]))
        pl.when(c == 1)(lambda: pltpu.sync_copy(accum.at[sl], c_hi.at[sl]))


@functools.partial(
    pl.kernel,
    out_type=(jax.ShapeDtypeStruct((E, H), jnp.float32),
              jax.ShapeDtypeStruct((E, H), jnp.float32)),
    mesh=_sc_mesh,
    scratch_types=[
        pltpu.VMEM((NCH, CW), jnp.int32),
        pltpu.VMEM((CW, H), jnp.float32),
        pltpu.SemaphoreType.DMA,
    ],
)
def _edge_gather_kernel(a_t, b_t, srcp, dstp, sa, sb, idx_v, rows_v, sem):
    c = lax.axis_index("c")
    s = lax.axis_index("s")

    def run(table, idxs, out):
        pltpu.sync_copy(idxs.at[s], idx_v)

        def chunk(j, carry):
            pltpu.async_copy(table.at[idx_v.at[j]], rows_v, sem).wait()
            pltpu.sync_copy(rows_v, out.at[pl.ds(s * ES + j * CW, CW)])
            return carry

        lax.fori_loop(0, NCH - 1, chunk, 0)
        pltpu.async_copy(table.at[idx_v.at[NCH - 1]], rows_v, sem).wait()
        pltpu.sync_copy(rows_v.at[pl.ds(0, TAIL)],
                        out.at[pl.ds(s * ES + (NCH - 1) * CW, TAIL)])

    pl.when(c == 0)(lambda: run(a_t, srcp, sa))
    pl.when(c == 1)(lambda: run(b_t, dstp, sb))


def _embed_body(x_ref, wlo_ref, whi_ref, blo_ref, bhi_ref, olo_ref, ohi_ref):
    x = x_ref[...]
    olo_ref[...] = jnp.dot(x, wlo_ref[...],
                           preferred_element_type=jnp.float32, precision=jax.lax.Precision.HIGHEST) + blo_ref[...]
    ohi_ref[...] = jnp.dot(x, whi_ref[...],
                           preferred_element_type=jnp.float32, precision=jax.lax.Precision.HIGHEST) + bhi_ref[...]


def _embed(x, W, b):
    R = 2000
    return pl.pallas_call(
        _embed_body,
        grid=(N // R,),
        in_specs=[
            pl.BlockSpec((R, DIN), lambda i: (i, 0)),
            pl.BlockSpec((DIN, HH), lambda i: (0, 0)),
            pl.BlockSpec((DIN, HH), lambda i: (0, 0)),
            pl.BlockSpec((1, HH), lambda i: (0, 0)),
            pl.BlockSpec((1, HH), lambda i: (0, 0)),
        ],
        out_specs=(pl.BlockSpec((R, HH), lambda i: (i, 0)),
                   pl.BlockSpec((R, HH), lambda i: (i, 0))),
        out_shape=(jax.ShapeDtypeStruct((N, HH), jnp.float32),
                   jax.ShapeDtypeStruct((N, HH), jnp.float32)),
    )(x, W[:, :HH], W[:, HH:], b[:HH].reshape(1, HH), b[HH:].reshape(1, HH))


def _layer_body(hlo_ref, hhi_ref, clo_ref, chi_ref, deg_ref,
                wtlo_ref, wthi_ref, wblo_ref, wbhi_ref,
                b_ref, bnsc_ref, bnsh_ref, olo_ref, ohi_ref):
    r = 1.0 / jnp.maximum(deg_ref[...][:, 0:1], 1.0)
    f32 = jnp.float32
    bundle = (jnp.dot(hlo_ref[...], wtlo_ref[...], preferred_element_type=f32, precision=_PREC)
              + jnp.dot(hhi_ref[...], wthi_ref[...], preferred_element_type=f32, precision=_PREC)
              + jnp.dot(clo_ref[...] * r, wblo_ref[...], preferred_element_type=f32, precision=_PREC)
              + jnp.dot(chi_ref[...] * r, wbhi_ref[...], preferred_element_type=f32, precision=_PREC)
              + b_ref[...])
    inv = 1.0 / jnp.maximum(
        jnp.sqrt(jnp.sum(bundle * bundle, axis=1, keepdims=True)), 1e-12)
    t = jnp.maximum(bundle * inv, 0.0) * bnsc_ref[...] + bnsh_ref[...]
    olo_ref[...] = hlo_ref[...] + t[:, :HH]
    ohi_ref[...] = hhi_ref[...] + t[:, HH:]


def _layer(h_lo, h_hi, c_lo, c_hi, deg16, Wt, Wb, b, bnsc, bnsh):
    R = 2000
    full = lambda i: (0, 0)
    blk = lambda i: (i, 0)
    return pl.pallas_call(
        _layer_body,
        grid=(N // R,),
        in_specs=[
            pl.BlockSpec((R, HH), blk), pl.BlockSpec((R, HH), blk),
            pl.BlockSpec((R, HH), blk), pl.BlockSpec((R, HH), blk),
            pl.BlockSpec((R, HH), blk),
            pl.BlockSpec((HH, H), full), pl.BlockSpec((HH, H), full),
            pl.BlockSpec((HH, H), full), pl.BlockSpec((HH, H), full),
            pl.BlockSpec((1, H), full), pl.BlockSpec((1, H), full),
            pl.BlockSpec((1, H), full),
        ],
        out_specs=(pl.BlockSpec((R, HH), blk), pl.BlockSpec((R, HH), blk)),
        out_shape=(jax.ShapeDtypeStruct((N, HH), jnp.float32),
                   jax.ShapeDtypeStruct((N, HH), jnp.float32)),
    )(h_lo, h_hi, c_lo, c_hi, deg16,
      Wt[:HH], Wt[HH:], Wb[:HH], Wb[HH:],
      b.reshape(1, H), bnsc.reshape(1, H), bnsh.reshape(1, H))


def _ab_body(hlo_ref, hhi_ref, walo_ref, wahi_ref, wblo_ref, wbhi_ref,
             a_ref, b_ref):
    f32 = jnp.float32
    hlo = hlo_ref[...]
    hhi = hhi_ref[...]
    a_ref[...] = (jnp.dot(hlo, walo_ref[...], preferred_element_type=f32, precision=_PREC)
                  + jnp.dot(hhi, wahi_ref[...], preferred_element_type=f32, precision=_PREC))
    b_ref[...] = (jnp.dot(hlo, wblo_ref[...], preferred_element_type=f32, precision=_PREC)
                  + jnp.dot(hhi, wbhi_ref[...], preferred_element_type=f32, precision=_PREC))


def _ab(h_lo, h_hi, Wa, Wb):
    R = 2000
    full = lambda i: (0, 0)
    blk = lambda i: (i, 0)
    return pl.pallas_call(
        _ab_body,
        grid=(N // R,),
        in_specs=[
            pl.BlockSpec((R, HH), blk), pl.BlockSpec((R, HH), blk),
            pl.BlockSpec((HH, H), full), pl.BlockSpec((HH, H), full),
            pl.BlockSpec((HH, H), full), pl.BlockSpec((HH, H), full),
        ],
        out_specs=(pl.BlockSpec((R, H), blk), pl.BlockSpec((R, H), blk)),
        out_shape=(jax.ShapeDtypeStruct((N, H), jnp.float32),
                   jax.ShapeDtypeStruct((N, H), jnp.float32)),
    )(h_lo, h_hi, Wa[:HH], Wa[HH:], Wb[:HH], Wb[HH:])


def _mlp_body(sa_ref, sb_ref, b0_ref, w1_ref, b1_ref, w2_ref, b2_ref, o_ref):
    f32 = jnp.float32
    y0 = jnp.maximum(sa_ref[...] + sb_ref[...] + b0_ref[...], 0.0)
    y1 = jnp.maximum(
        jnp.dot(y0, w1_ref[...], preferred_element_type=f32, precision=_PREC) + b1_ref[...], 0.0)
    o_ref[...] = jnp.dot(y1, w2_ref[...], preferred_element_type=f32, precision=_PREC) + b2_ref[...]


def _mlp(sa, sb, b0, W1, b1, W2, b2):
    R = 2000
    full = lambda i: (0, 0)
    blk = lambda i: (i, 0)
    return pl.pallas_call(
        _mlp_body,
        grid=(E // R,),
        in_specs=[
            pl.BlockSpec((R, H), blk), pl.BlockSpec((R, H), blk),
            pl.BlockSpec((1, H), full),
            pl.BlockSpec((H, HH), full), pl.BlockSpec((1, HH), full),
            pl.BlockSpec((HH, 2), full), pl.BlockSpec((1, 2), full),
        ],
        out_specs=pl.BlockSpec((R, 2), blk),
        out_shape=jax.ShapeDtypeStruct((E, 2), jnp.float32),
    )(sa, sb, b0.reshape(1, H), W1, b1.reshape(1, HH), W2, b2.reshape(1, 2))


def kernel(x, edge_index, W_embed, b_embed,
           W_sage0, b_sage0, bn_g0, bn_b0, bn_m0, bn_v0,
           W_sage1, b_sage1, bn_g1, bn_b1, bn_m1, bn_v1,
           W_sage2, b_sage2, bn_g2, bn_b2, bn_m2, bn_v2,
           W_mlp0, b_mlp0, W_mlp1, b_mlp1, W_mlp2, b_mlp2):
    src = edge_index[0]
    dst = edge_index[1]
    srcp = jnp.pad(src.reshape(NSUB, ES),
                   ((0, 0), (0, PADE - ES))).reshape(NSUB, NCH, CW)
    dstp = jnp.pad(dst.reshape(NSUB, ES), ((0, 0), (0, PADE - ES)),
                   constant_values=N).reshape(NSUB, NCH, CW)
    zeros = jnp.zeros((CPR, HH), jnp.float32)
    ones_cw = jnp.ones((CW, HH), jnp.float32)

    deg16 = _deg_kernel(dstp, ones_cw, zeros)
    h_lo, h_hi = _embed(x, W_embed, b_embed)

    sages = ((W_sage0, b_sage0, bn_g0, bn_b0, bn_m0, bn_v0),
             (W_sage1, b_sage1, bn_g1, bn_b1, bn_m1, bn_v1),
             (W_sage2, b_sage2, bn_g2, bn_b2, bn_m2, bn_v2))
    for W, b, g, bb, m, v in sages:
        c_lo, c_hi = _seg_kernel(h_lo, h_hi, srcp, dstp, zeros)
        bnsc = g * jax.lax.rsqrt(v + BN_EPS)
        bnsh = bb - m * bnsc
        h_lo, h_hi = _layer(h_lo, h_hi, c_lo, c_hi, deg16,
                            W[:H], W[H:], b, bnsc, bnsh)

    a_t, b_t = _ab(h_lo, h_hi, W_mlp0[:H], W_mlp0[H:])
    sa, sb = _edge_gather_kernel(a_t, b_t, srcp, dstp)
    return _mlp(sa, sb, b_mlp0, W_mlp1, b_mlp1, W_mlp2, b_mlp2)


# speedup vs baseline: 3.1328x; 1.1142x over previous
import functools

import jax
import jax.numpy as jnp
from jax import lax
from jax.experimental import pallas as pl
from jax.experimental.pallas import tpu as pltpu
from jax.experimental.pallas import tpu_sc as plsc

N = 10000
E = 160000
DIN = 1024
H = 256
HH = 128
NSUB = 16
ES = E // NSUB
CW = 128
NCH = (ES + CW - 1) // CW
TAIL = ES - (NCH - 1) * CW
PADE = NCH * CW
NDUM = 8
NCOPY = 10
CPR = N // NCOPY
BN_EPS = 1e-5

_PREC = jax.lax.Precision.HIGHEST

_sc_mesh = plsc.VectorSubcoreMesh(core_axis_name="c", subcore_axis_name="s",
                                  num_cores=2, num_subcores=NSUB)


@functools.partial(
    pl.kernel,
    out_type=jax.ShapeDtypeStruct((N, HH), jnp.float32),
    mesh=_sc_mesh,
    scratch_types=[
        pltpu.VMEM((NCH, CW), jnp.int32),
        pltpu.VMEM((CW, HH), jnp.float32),
        pltpu.VMEM_SHARED((N + NDUM, HH), jnp.float32),
        pltpu.SemaphoreType.DMA,
    ],
)
def _deg_kernel(dstp, ones, zeros, deg_out, dst_v, ones_v, accum, sem):
    c = lax.axis_index("c")
    s = lax.axis_index("s")

    @pl.when(c == 0)
    def _():
        pl.when(s < NCOPY)(
            lambda: pltpu.sync_copy(zeros, accum.at[pl.ds(s * CPR, CPR)]))
        pltpu.sync_copy(dstp.at[s], dst_v)
        pltpu.sync_copy(ones, ones_v)
        plsc.subcore_barrier()

        def chunk(j, carry):
            pltpu.sync_copy(ones_v, accum.at[dst_v.at[j]], add=True)
            return carry

        lax.fori_loop(0, NCH, chunk, 0)
        plsc.subcore_barrier()
        sl = pl.ds(s * CPR, CPR)
        pl.when(s < NCOPY)(
            lambda: pltpu.sync_copy(accum.at[sl], deg_out.at[sl]))


@functools.partial(
    pl.kernel,
    out_type=(jax.ShapeDtypeStruct((N, HH), jnp.float32),
              jax.ShapeDtypeStruct((N, HH), jnp.float32)),
    mesh=_sc_mesh,
    scratch_types=[
        pltpu.VMEM((2, CW), jnp.int32),
        pltpu.VMEM((2, CW), jnp.int32),
        pltpu.VMEM((2, CW, HH), jnp.float32),
        pltpu.VMEM_SHARED((N + NDUM, HH), jnp.float32),
        pltpu.SemaphoreType.DMA,
        pltpu.SemaphoreType.DMA,
    ],
)
def _seg_kernel(h_lo, h_hi, srcp, dstp, zeros, c_lo, c_hi,
                src_v, dst_v, rows_v, accum, sem, sem_i):
    c = lax.axis_index("c")
    s = lax.axis_index("s")
    pl.when(s < NCOPY)(
        lambda: pltpu.sync_copy(zeros, accum.at[pl.ds(s * CPR, CPR)]))
    pltpu.sync_copy(srcp.at[s, 0], src_v.at[0])
    pltpu.sync_copy(dstp.at[s, 0], dst_v.at[0])
    plsc.subcore_barrier()

    def run(table):
        pltpu.async_copy(srcp.at[s, 1], src_v.at[1], sem_i)
        pltpu.async_copy(dstp.at[s, 1], dst_v.at[1], sem_i)
        pltpu.async_copy(table.at[src_v.at[0]], rows_v.at[0], sem)

        def chunk(j, carry):
            p = lax.rem(j, 2)
            pltpu.make_async_copy(table.at[src_v.at[p]],
                                  rows_v.at[p], sem).wait()

            @pl.when(j + 1 < NCH)
            def _():
                pltpu.make_async_copy(srcp.at[s, j + 1],
                                      src_v.at[1 - p], sem_i).wait()
                pltpu.make_async_copy(dstp.at[s, j + 1],
                                      dst_v.at[1 - p], sem_i).wait()
                pltpu.async_copy(
                    table.at[src_v.at[1 - p]], rows_v.at[1 - p], sem)
            pltpu.sync_copy(rows_v.at[p], accum.at[dst_v.at[p]], add=True)

            @pl.when(j + 2 < NCH)
            def _():
                pltpu.async_copy(srcp.at[s, j + 2], src_v.at[p], sem_i)
                pltpu.async_copy(dstp.at[s, j + 2], dst_v.at[p], sem_i)
            return carry

        lax.fori_loop(0, NCH, chunk, 0)

    pl.when(c == 0)(lambda: run(h_lo))
    pl.when(c == 1)(lambda: run(h_hi))
    plsc.subcore_barrier()
    sl = pl.ds(s * CPR, CPR)

    @pl.when(s < NCOPY)
    def _():
        pl.when(c == 0)(lambda: pltpu.sync_copy(accum.at[sl], c_lo.at[sl]))
        pl.when(c == 1)(lambda: pltpu.sync_copy(accum.at[sl], c_hi.at[sl]))


@functools.partial(
    pl.kernel,
    out_type=(jax.ShapeDtypeStruct((E, H), jnp.float32),
              jax.ShapeDtypeStruct((E, H), jnp.float32)),
    mesh=_sc_mesh,
    scratch_types=[
        pltpu.VMEM((NCH, CW), jnp.int32),
        pltpu.VMEM((2, CW, H), jnp.float32),
        pltpu.SemaphoreType.DMA,
    ],
)
def _edge_gather_kernel(a_t, b_t, srcp, dstp, sa, sb, idx_v, rows_v, sem):
    c = lax.axis_index("c")
    s = lax.axis_index("s")

    def run(table, idxs, out):
        pltpu.sync_copy(idxs.at[s], idx_v)
        pltpu.async_copy(table.at[idx_v.at[0]], rows_v.at[0], sem)

        def chunk(j, carry):
            p = lax.rem(j, 2)
            pltpu.make_async_copy(table.at[idx_v.at[j]],
                                  rows_v.at[p], sem).wait()
            pltpu.async_copy(table.at[idx_v.at[j + 1]], rows_v.at[1 - p], sem)
            pltpu.sync_copy(rows_v.at[p], out.at[pl.ds(s * ES + j * CW, CW)])
            return carry

        lax.fori_loop(0, NCH - 1, chunk, 0)
        p_last = (NCH - 1) % 2
        pltpu.make_async_copy(table.at[idx_v.at[NCH - 1]],
                              rows_v.at[p_last], sem).wait()
        pltpu.sync_copy(rows_v.at[p_last].at[pl.ds(0, TAIL)],
                        out.at[pl.ds(s * ES + (NCH - 1) * CW, TAIL)])

    pl.when(c == 0)(lambda: run(a_t, srcp, sa))
    pl.when(c == 1)(lambda: run(b_t, dstp, sb))


def _embed_body(x_ref, wlo_ref, whi_ref, blo_ref, bhi_ref, olo_ref, ohi_ref):
    x = x_ref[...]
    olo_ref[...] = jnp.dot(x, wlo_ref[...],
                           preferred_element_type=jnp.float32, precision=jax.lax.Precision.HIGHEST) + blo_ref[...]
    ohi_ref[...] = jnp.dot(x, whi_ref[...],
                           preferred_element_type=jnp.float32, precision=jax.lax.Precision.HIGHEST) + bhi_ref[...]


def _embed(x, W, b):
    R = 2000
    return pl.pallas_call(
        _embed_body,
        grid=(N // R,),
        in_specs=[
            pl.BlockSpec((R, DIN), lambda i: (i, 0)),
            pl.BlockSpec((DIN, HH), lambda i: (0, 0)),
            pl.BlockSpec((DIN, HH), lambda i: (0, 0)),
            pl.BlockSpec((1, HH), lambda i: (0, 0)),
            pl.BlockSpec((1, HH), lambda i: (0, 0)),
        ],
        out_specs=(pl.BlockSpec((R, HH), lambda i: (i, 0)),
                   pl.BlockSpec((R, HH), lambda i: (i, 0))),
        out_shape=(jax.ShapeDtypeStruct((N, HH), jnp.float32),
                   jax.ShapeDtypeStruct((N, HH), jnp.float32)),
    )(x, W[:, :HH], W[:, HH:], b[:HH].reshape(1, HH), b[HH:].reshape(1, HH))


def _layer_body(hlo_ref, hhi_ref, clo_ref, chi_ref, deg_ref,
                wtlo_ref, wthi_ref, wblo_ref, wbhi_ref,
                b_ref, bnsc_ref, bnsh_ref, olo_ref, ohi_ref):
    r = 1.0 / jnp.maximum(deg_ref[...][:, 0:1], 1.0)
    f32 = jnp.float32
    bundle = (jnp.dot(hlo_ref[...], wtlo_ref[...], preferred_element_type=f32, precision=_PREC)
              + jnp.dot(hhi_ref[...], wthi_ref[...], preferred_element_type=f32, precision=_PREC)
              + jnp.dot(clo_ref[...] * r, wblo_ref[...], preferred_element_type=f32, precision=_PREC)
              + jnp.dot(chi_ref[...] * r, wbhi_ref[...], preferred_element_type=f32, precision=_PREC)
              + b_ref[...])
    inv = 1.0 / jnp.maximum(
        jnp.sqrt(jnp.sum(bundle * bundle, axis=1, keepdims=True)), 1e-12)
    t = jnp.maximum(bundle * inv, 0.0) * bnsc_ref[...] + bnsh_ref[...]
    olo_ref[...] = hlo_ref[...] + t[:, :HH]
    ohi_ref[...] = hhi_ref[...] + t[:, HH:]


def _layer(h_lo, h_hi, c_lo, c_hi, deg16, Wt, Wb, b, bnsc, bnsh):
    R = 2000
    full = lambda i: (0, 0)
    blk = lambda i: (i, 0)
    return pl.pallas_call(
        _layer_body,
        grid=(N // R,),
        in_specs=[
            pl.BlockSpec((R, HH), blk), pl.BlockSpec((R, HH), blk),
            pl.BlockSpec((R, HH), blk), pl.BlockSpec((R, HH), blk),
            pl.BlockSpec((R, HH), blk),
            pl.BlockSpec((HH, H), full), pl.BlockSpec((HH, H), full),
            pl.BlockSpec((HH, H), full), pl.BlockSpec((HH, H), full),
            pl.BlockSpec((1, H), full), pl.BlockSpec((1, H), full),
            pl.BlockSpec((1, H), full),
        ],
        out_specs=(pl.BlockSpec((R, HH), blk), pl.BlockSpec((R, HH), blk)),
        out_shape=(jax.ShapeDtypeStruct((N, HH), jnp.float32),
                   jax.ShapeDtypeStruct((N, HH), jnp.float32)),
    )(h_lo, h_hi, c_lo, c_hi, deg16,
      Wt[:HH], Wt[HH:], Wb[:HH], Wb[HH:],
      b.reshape(1, H), bnsc.reshape(1, H), bnsh.reshape(1, H))


def _ab_body(hlo_ref, hhi_ref, walo_ref, wahi_ref, wblo_ref, wbhi_ref,
             a_ref, b_ref):
    f32 = jnp.float32
    hlo = hlo_ref[...]
    hhi = hhi_ref[...]
    a_ref[...] = (jnp.dot(hlo, walo_ref[...], preferred_element_type=f32, precision=_PREC)
                  + jnp.dot(hhi, wahi_ref[...], preferred_element_type=f32, precision=_PREC))
    b_ref[...] = (jnp.dot(hlo, wblo_ref[...], preferred_element_type=f32, precision=_PREC)
                  + jnp.dot(hhi, wbhi_ref[...], preferred_element_type=f32, precision=_PREC))


def _ab(h_lo, h_hi, Wa, Wb):
    R = 2000
    full = lambda i: (0, 0)
    blk = lambda i: (i, 0)
    return pl.pallas_call(
        _ab_body,
        grid=(N // R,),
        in_specs=[
            pl.BlockSpec((R, HH), blk), pl.BlockSpec((R, HH), blk),
            pl.BlockSpec((HH, H), full), pl.BlockSpec((HH, H), full),
            pl.BlockSpec((HH, H), full), pl.BlockSpec((HH, H), full),
        ],
        out_specs=(pl.BlockSpec((R, H), blk), pl.BlockSpec((R, H), blk)),
        out_shape=(jax.ShapeDtypeStruct((N, H), jnp.float32),
                   jax.ShapeDtypeStruct((N, H), jnp.float32)),
    )(h_lo, h_hi, Wa[:HH], Wa[HH:], Wb[:HH], Wb[HH:])


def _mlp_body(sa_ref, sb_ref, b0_ref, w1_ref, b1_ref, w2_ref, b2_ref, o_ref):
    f32 = jnp.float32
    y0 = jnp.maximum(sa_ref[...] + sb_ref[...] + b0_ref[...], 0.0)
    y1 = jnp.maximum(
        jnp.dot(y0, w1_ref[...], preferred_element_type=f32, precision=_PREC) + b1_ref[...], 0.0)
    o_ref[...] = jnp.dot(y1, w2_ref[...], preferred_element_type=f32, precision=_PREC) + b2_ref[...]


def _mlp(sa, sb, b0, W1, b1, W2, b2):
    R = 2000
    full = lambda i: (0, 0)
    blk = lambda i: (i, 0)
    return pl.pallas_call(
        _mlp_body,
        grid=(E // R,),
        in_specs=[
            pl.BlockSpec((R, H), blk), pl.BlockSpec((R, H), blk),
            pl.BlockSpec((1, H), full),
            pl.BlockSpec((H, HH), full), pl.BlockSpec((1, HH), full),
            pl.BlockSpec((HH, 2), full), pl.BlockSpec((1, 2), full),
        ],
        out_specs=pl.BlockSpec((R, 2), blk),
        out_shape=jax.ShapeDtypeStruct((E, 2), jnp.float32),
    )(sa, sb, b0.reshape(1, H), W1, b1.reshape(1, HH), W2, b2.reshape(1, 2))


def kernel(x, edge_index, W_embed, b_embed,
           W_sage0, b_sage0, bn_g0, bn_b0, bn_m0, bn_v0,
           W_sage1, b_sage1, bn_g1, bn_b1, bn_m1, bn_v1,
           W_sage2, b_sage2, bn_g2, bn_b2, bn_m2, bn_v2,
           W_mlp0, b_mlp0, W_mlp1, b_mlp1, W_mlp2, b_mlp2):
    src = edge_index[0]
    dst = edge_index[1]
    srcp = jnp.pad(src.reshape(NSUB, ES),
                   ((0, 0), (0, PADE - ES))).reshape(NSUB, NCH, CW)
    dstp = jnp.pad(dst.reshape(NSUB, ES), ((0, 0), (0, PADE - ES)),
                   constant_values=N).reshape(NSUB, NCH, CW)
    zeros = jnp.zeros((CPR, HH), jnp.float32)
    ones_cw = jnp.ones((CW, HH), jnp.float32)

    deg16 = _deg_kernel(dstp, ones_cw, zeros)
    h_lo, h_hi = _embed(x, W_embed, b_embed)

    sages = ((W_sage0, b_sage0, bn_g0, bn_b0, bn_m0, bn_v0),
             (W_sage1, b_sage1, bn_g1, bn_b1, bn_m1, bn_v1),
             (W_sage2, b_sage2, bn_g2, bn_b2, bn_m2, bn_v2))
    for W, b, g, bb, m, v in sages:
        c_lo, c_hi = _seg_kernel(h_lo, h_hi, srcp, dstp, zeros)
        bnsc = g * jax.lax.rsqrt(v + BN_EPS)
        bnsh = bb - m * bnsc
        h_lo, h_hi = _layer(h_lo, h_hi, c_lo, c_hi, deg16,
                            W[:H], W[H:], b, bnsc, bnsh)

    a_t, b_t = _ab(h_lo, h_hi, W_mlp0[:H], W_mlp0[H:])
    sa, sb = _edge_gather_kernel(a_t, b_t, srcp, dstp)
    return _mlp(sa, sb, b_mlp0, W_mlp1, b_mlp1, W_mlp2, b_mlp2)


# speedup vs baseline: 3.3614x; 1.0730x over previous
import functools

import jax
import jax.numpy as jnp
from jax import lax
from jax.experimental import pallas as pl
from jax.experimental.pallas import tpu as pltpu
from jax.experimental.pallas import tpu_sc as plsc

N = 10000
E = 160000
DIN = 1024
H = 256
HH = 128
NSUB = 16
ES = E // NSUB
CW = 128
NCH = (ES + CW - 1) // CW
TAIL = ES - (NCH - 1) * CW
PADE = NCH * CW
NDUM = 8
NCOPY = 10
CPR = N // NCOPY
BN_EPS = 1e-5

_PREC = jax.lax.Precision.HIGHEST

_sc_mesh = plsc.VectorSubcoreMesh(core_axis_name="c", subcore_axis_name="s",
                                  num_cores=2, num_subcores=NSUB)


@functools.partial(
    pl.kernel,
    out_type=jax.ShapeDtypeStruct((N, HH), jnp.float32),
    mesh=_sc_mesh,
    scratch_types=[
        pltpu.VMEM((NCH, CW), jnp.int32),
        pltpu.VMEM((CW, HH), jnp.float32),
        pltpu.VMEM_SHARED((N + NDUM, HH), jnp.float32),
        pltpu.SemaphoreType.DMA,
    ],
)
def _deg_kernel(dstp, ones, zeros, deg_out, dst_v, ones_v, accum, sem):
    c = lax.axis_index("c")
    s = lax.axis_index("s")

    @pl.when(c == 0)
    def _():
        pl.when(s < NCOPY)(
            lambda: pltpu.sync_copy(zeros, accum.at[pl.ds(s * CPR, CPR)]))
        pltpu.sync_copy(dstp.at[s], dst_v)
        pltpu.sync_copy(ones, ones_v)
        plsc.subcore_barrier()

        def chunk(j, carry):
            pltpu.sync_copy(ones_v, accum.at[dst_v.at[j]], add=True)
            return carry

        lax.fori_loop(0, NCH, chunk, 0)
        plsc.subcore_barrier()
        sl = pl.ds(s * CPR, CPR)
        pl.when(s < NCOPY)(
            lambda: pltpu.sync_copy(accum.at[sl], deg_out.at[sl]))


@functools.partial(
    pl.kernel,
    out_type=(jax.ShapeDtypeStruct((N, HH), jnp.float32),
              jax.ShapeDtypeStruct((N, HH), jnp.float32)),
    mesh=_sc_mesh,
    scratch_types=[
        pltpu.VMEM((2, CW), jnp.int32),
        pltpu.VMEM((2, CW), jnp.int32),
        pltpu.VMEM((2, CW, HH), jnp.float32),
        pltpu.VMEM_SHARED((N + NDUM, HH), jnp.float32),
        pltpu.SemaphoreType.DMA,
        pltpu.SemaphoreType.DMA,
    ],
)
def _seg_kernel(h_lo, h_hi, srcp, dstp, zeros, c_lo, c_hi,
                src_v, dst_v, rows_v, accum, sem, sem_i):
    c = lax.axis_index("c")
    s = lax.axis_index("s")
    pl.when(s < NCOPY)(
        lambda: pltpu.sync_copy(zeros, accum.at[pl.ds(s * CPR, CPR)]))
    pltpu.sync_copy(srcp.at[s, 0], src_v.at[0])
    pltpu.sync_copy(dstp.at[s, 0], dst_v.at[0])
    plsc.subcore_barrier()

    def run(table):
        pltpu.async_copy(srcp.at[s, 1], src_v.at[1], sem_i)
        pltpu.async_copy(dstp.at[s, 1], dst_v.at[1], sem_i)
        pltpu.async_copy(table.at[src_v.at[0]], rows_v.at[0], sem)

        def chunk(j, carry):
            p = lax.rem(j, 2)
            pltpu.make_async_copy(table.at[src_v.at[p]],
                                  rows_v.at[p], sem).wait()

            @pl.when(j + 1 < NCH)
            def _():
                pltpu.make_async_copy(srcp.at[s, j + 1],
                                      src_v.at[1 - p], sem_i).wait()
                pltpu.make_async_copy(dstp.at[s, j + 1],
                                      dst_v.at[1 - p], sem_i).wait()
                pltpu.async_copy(
                    table.at[src_v.at[1 - p]], rows_v.at[1 - p], sem)
            pltpu.sync_copy(rows_v.at[p], accum.at[dst_v.at[p]], add=True)

            @pl.when(j + 2 < NCH)
            def _():
                pltpu.async_copy(srcp.at[s, j + 2], src_v.at[p], sem_i)
                pltpu.async_copy(dstp.at[s, j + 2], dst_v.at[p], sem_i)
            return carry

        lax.fori_loop(0, NCH, chunk, 0)

    pl.when(c == 0)(lambda: run(h_lo))
    pl.when(c == 1)(lambda: run(h_hi))
    plsc.subcore_barrier()
    sl = pl.ds(s * CPR, CPR)

    @pl.when(s < NCOPY)
    def _():
        pl.when(c == 0)(lambda: pltpu.sync_copy(accum.at[sl], c_lo.at[sl]))
        pl.when(c == 1)(lambda: pltpu.sync_copy(accum.at[sl], c_hi.at[sl]))


@functools.partial(
    pl.kernel,
    out_type=(jax.ShapeDtypeStruct((E, HH), jnp.float32),
              jax.ShapeDtypeStruct((E, HH), jnp.float32)),
    mesh=_sc_mesh,
    scratch_types=[
        pltpu.VMEM((NCH, CW), jnp.int32),
        pltpu.VMEM((4, CW, HH), jnp.float32),
        pltpu.SemaphoreType.DMA,
    ],
)
def _edge_gather_kernel(a_t, b_t, srcp, dstp, sa, sb, idx_v, rows_v, sem):
    c = lax.axis_index("c")
    s = lax.axis_index("s")

    def run(table, idxs, out):
        pltpu.sync_copy(idxs.at[s], idx_v)
        for k in range(3):
            pltpu.async_copy(table.at[idx_v.at[k]], rows_v.at[k], sem)

        def chunk(j, carry):
            p = lax.rem(j, 4)
            pltpu.make_async_copy(table.at[idx_v.at[j]],
                                  rows_v.at[p], sem).wait()

            @pl.when(j + 3 < NCH)
            def _():
                pltpu.async_copy(table.at[idx_v.at[j + 3]],
                                 rows_v.at[lax.rem(j + 3, 4)], sem)
            pltpu.sync_copy(rows_v.at[p], out.at[pl.ds(s * ES + j * CW, CW)])
            return carry

        lax.fori_loop(0, NCH - 1, chunk, 0)
        p_last = (NCH - 1) % 4
        pltpu.make_async_copy(table.at[idx_v.at[NCH - 1]],
                              rows_v.at[p_last], sem).wait()
        pltpu.sync_copy(rows_v.at[p_last].at[pl.ds(0, TAIL)],
                        out.at[pl.ds(s * ES + (NCH - 1) * CW, TAIL)])

    pl.when(c == 0)(lambda: run(a_t, srcp, sa))
    pl.when(c == 1)(lambda: run(b_t, dstp, sb))


def _embed_body(x_ref, wlo_ref, whi_ref, blo_ref, bhi_ref, olo_ref, ohi_ref):
    x = x_ref[...]
    olo_ref[...] = jnp.dot(x, wlo_ref[...],
                           preferred_element_type=jnp.float32, precision=jax.lax.Precision.HIGHEST) + blo_ref[...]
    ohi_ref[...] = jnp.dot(x, whi_ref[...],
                           preferred_element_type=jnp.float32, precision=jax.lax.Precision.HIGHEST) + bhi_ref[...]


def _embed(x, W, b):
    R = 2000
    return pl.pallas_call(
        _embed_body,
        grid=(N // R,),
        in_specs=[
            pl.BlockSpec((R, DIN), lambda i: (i, 0)),
            pl.BlockSpec((DIN, HH), lambda i: (0, 0)),
            pl.BlockSpec((DIN, HH), lambda i: (0, 0)),
            pl.BlockSpec((1, HH), lambda i: (0, 0)),
            pl.BlockSpec((1, HH), lambda i: (0, 0)),
        ],
        out_specs=(pl.BlockSpec((R, HH), lambda i: (i, 0)),
                   pl.BlockSpec((R, HH), lambda i: (i, 0))),
        out_shape=(jax.ShapeDtypeStruct((N, HH), jnp.float32),
                   jax.ShapeDtypeStruct((N, HH), jnp.float32)),
    )(x, W[:, :HH], W[:, HH:], b[:HH].reshape(1, HH), b[HH:].reshape(1, HH))


def _layer_body(hlo_ref, hhi_ref, clo_ref, chi_ref, deg_ref,
                wtlo_ref, wthi_ref, wblo_ref, wbhi_ref,
                b_ref, bnsc_ref, bnsh_ref, olo_ref, ohi_ref):
    r = 1.0 / jnp.maximum(deg_ref[...][:, 0:1], 1.0)
    f32 = jnp.float32
    bundle = (jnp.dot(hlo_ref[...], wtlo_ref[...], preferred_element_type=f32, precision=_PREC)
              + jnp.dot(hhi_ref[...], wthi_ref[...], preferred_element_type=f32, precision=_PREC)
              + jnp.dot(clo_ref[...] * r, wblo_ref[...], preferred_element_type=f32, precision=_PREC)
              + jnp.dot(chi_ref[...] * r, wbhi_ref[...], preferred_element_type=f32, precision=_PREC)
              + b_ref[...])
    inv = 1.0 / jnp.maximum(
        jnp.sqrt(jnp.sum(bundle * bundle, axis=1, keepdims=True)), 1e-12)
    t = jnp.maximum(bundle * inv, 0.0) * bnsc_ref[...] + bnsh_ref[...]
    olo_ref[...] = hlo_ref[...] + t[:, :HH]
    ohi_ref[...] = hhi_ref[...] + t[:, HH:]


def _layer(h_lo, h_hi, c_lo, c_hi, deg16, Wt, Wb, b, bnsc, bnsh):
    R = 2000
    full = lambda i: (0, 0)
    blk = lambda i: (i, 0)
    return pl.pallas_call(
        _layer_body,
        grid=(N // R,),
        in_specs=[
            pl.BlockSpec((R, HH), blk), pl.BlockSpec((R, HH), blk),
            pl.BlockSpec((R, HH), blk), pl.BlockSpec((R, HH), blk),
            pl.BlockSpec((R, HH), blk),
            pl.BlockSpec((HH, H), full), pl.BlockSpec((HH, H), full),
            pl.BlockSpec((HH, H), full), pl.BlockSpec((HH, H), full),
            pl.BlockSpec((1, H), full), pl.BlockSpec((1, H), full),
            pl.BlockSpec((1, H), full),
        ],
        out_specs=(pl.BlockSpec((R, HH), blk), pl.BlockSpec((R, HH), blk)),
        out_shape=(jax.ShapeDtypeStruct((N, HH), jnp.float32),
                   jax.ShapeDtypeStruct((N, HH), jnp.float32)),
    )(h_lo, h_hi, c_lo, c_hi, deg16,
      Wt[:HH], Wt[HH:], Wb[:HH], Wb[HH:],
      b.reshape(1, H), bnsc.reshape(1, H), bnsh.reshape(1, H))


def _pack_bf16(v):
    vb = v.astype(jnp.bfloat16)
    lo = lax.bitcast_convert_type(vb[:, :HH], jnp.uint16).astype(jnp.uint32)
    hi = lax.bitcast_convert_type(vb[:, HH:], jnp.uint16).astype(jnp.uint32)
    return lax.bitcast_convert_type(lo | (hi << 16), jnp.float32)


def _unpack_bf16(w):
    u = lax.bitcast_convert_type(w, jnp.uint32)
    lo = lax.bitcast_convert_type((u & 0xFFFF).astype(jnp.uint16),
                                  jnp.bfloat16).astype(jnp.float32)
    hi = lax.bitcast_convert_type((u >> 16).astype(jnp.uint16),
                                  jnp.bfloat16).astype(jnp.float32)
    return lo, hi


def _ab_body(hlo_ref, hhi_ref, walo_ref, wahi_ref, wblo_ref, wbhi_ref,
             a_ref, b_ref):
    f32 = jnp.float32
    hlo = hlo_ref[...]
    hhi = hhi_ref[...]
    a = (jnp.dot(hlo, walo_ref[...], preferred_element_type=f32, precision=_PREC)
         + jnp.dot(hhi, wahi_ref[...], preferred_element_type=f32, precision=_PREC))
    b = (jnp.dot(hlo, wblo_ref[...], preferred_element_type=f32, precision=_PREC)
         + jnp.dot(hhi, wbhi_ref[...], preferred_element_type=f32, precision=_PREC))
    a_ref[...] = _pack_bf16(a)
    b_ref[...] = _pack_bf16(b)


def _ab(h_lo, h_hi, Wa, Wb):
    R = 2000
    full = lambda i: (0, 0)
    blk = lambda i: (i, 0)
    return pl.pallas_call(
        _ab_body,
        grid=(N // R,),
        in_specs=[
            pl.BlockSpec((R, HH), blk), pl.BlockSpec((R, HH), blk),
            pl.BlockSpec((HH, H), full), pl.BlockSpec((HH, H), full),
            pl.BlockSpec((HH, H), full), pl.BlockSpec((HH, H), full),
        ],
        out_specs=(pl.BlockSpec((R, HH), blk), pl.BlockSpec((R, HH), blk)),
        out_shape=(jax.ShapeDtypeStruct((N, HH), jnp.float32),
                   jax.ShapeDtypeStruct((N, HH), jnp.float32)),
    )(h_lo, h_hi, Wa[:HH], Wa[HH:], Wb[:HH], Wb[HH:])


def _mlp_body(sa_ref, sb_ref, b0lo_ref, b0hi_ref,
              w1lo_ref, w1hi_ref, b1_ref, w2_ref, b2_ref, o_ref):
    f32 = jnp.float32
    sa_lo, sa_hi = _unpack_bf16(sa_ref[...])
    sb_lo, sb_hi = _unpack_bf16(sb_ref[...])
    y0_lo = jnp.maximum(sa_lo + sb_lo + b0lo_ref[...], 0.0)
    y0_hi = jnp.maximum(sa_hi + sb_hi + b0hi_ref[...], 0.0)
    y1 = jnp.maximum(
        jnp.dot(y0_lo, w1lo_ref[...], preferred_element_type=f32, precision=_PREC)
        + jnp.dot(y0_hi, w1hi_ref[...], preferred_element_type=f32, precision=_PREC)
        + b1_ref[...], 0.0)
    o_ref[...] = jnp.dot(y1, w2_ref[...], preferred_element_type=f32, precision=_PREC) + b2_ref[...]


def _mlp(sa, sb, b0, W1, b1, W2, b2):
    R = 2000
    full = lambda i: (0, 0)
    blk = lambda i: (i, 0)
    return pl.pallas_call(
        _mlp_body,
        grid=(E // R,),
        in_specs=[
            pl.BlockSpec((R, HH), blk), pl.BlockSpec((R, HH), blk),
            pl.BlockSpec((1, HH), full), pl.BlockSpec((1, HH), full),
            pl.BlockSpec((HH, HH), full), pl.BlockSpec((HH, HH), full),
            pl.BlockSpec((1, HH), full),
            pl.BlockSpec((HH, 2), full), pl.BlockSpec((1, 2), full),
        ],
        out_specs=pl.BlockSpec((R, 2), blk),
        out_shape=jax.ShapeDtypeStruct((E, 2), jnp.float32),
    )(sa, sb, b0[:HH].reshape(1, HH), b0[HH:].reshape(1, HH),
      W1[:HH], W1[HH:], b1.reshape(1, HH), W2, b2.reshape(1, 2))


def kernel(x, edge_index, W_embed, b_embed,
           W_sage0, b_sage0, bn_g0, bn_b0, bn_m0, bn_v0,
           W_sage1, b_sage1, bn_g1, bn_b1, bn_m1, bn_v1,
           W_sage2, b_sage2, bn_g2, bn_b2, bn_m2, bn_v2,
           W_mlp0, b_mlp0, W_mlp1, b_mlp1, W_mlp2, b_mlp2):
    src = edge_index[0]
    dst = edge_index[1]
    srcp = jnp.pad(src.reshape(NSUB, ES),
                   ((0, 0), (0, PADE - ES))).reshape(NSUB, NCH, CW)
    dstp = jnp.pad(dst.reshape(NSUB, ES), ((0, 0), (0, PADE - ES)),
                   constant_values=N).reshape(NSUB, NCH, CW)
    zeros = jnp.zeros((CPR, HH), jnp.float32)
    ones_cw = jnp.ones((CW, HH), jnp.float32)

    deg16 = _deg_kernel(dstp, ones_cw, zeros)
    h_lo, h_hi = _embed(x, W_embed, b_embed)

    sages = ((W_sage0, b_sage0, bn_g0, bn_b0, bn_m0, bn_v0),
             (W_sage1, b_sage1, bn_g1, bn_b1, bn_m1, bn_v1),
             (W_sage2, b_sage2, bn_g2, bn_b2, bn_m2, bn_v2))
    for W, b, g, bb, m, v in sages:
        c_lo, c_hi = _seg_kernel(h_lo, h_hi, srcp, dstp, zeros)
        bnsc = g * jax.lax.rsqrt(v + BN_EPS)
        bnsh = bb - m * bnsc
        h_lo, h_hi = _layer(h_lo, h_hi, c_lo, c_hi, deg16,
                            W[:H], W[H:], b, bnsc, bnsh)

    a_t, b_t = _ab(h_lo, h_hi, W_mlp0[:H], W_mlp0[H:])
    sa, sb = _edge_gather_kernel(a_t, b_t, srcp, dstp)
    return _mlp(sa, sb, b_mlp0, W_mlp1, b_mlp1, W_mlp2, b_mlp2)


# speedup vs baseline: 3.6166x; 1.0759x over previous
import functools

import jax
import jax.numpy as jnp
from jax import lax
from jax.experimental import pallas as pl
from jax.experimental.pallas import tpu as pltpu
from jax.experimental.pallas import tpu_sc as plsc

N = 10000
E = 160000
DIN = 1024
H = 256
HH = 128
NSUB = 16
ES = E // NSUB
CW = 128
NCH = (ES + CW - 1) // CW
TAIL = ES - (NCH - 1) * CW
PADE = NCH * CW
NDUM = 8
NCOPY = 10
CPR = N // NCOPY
BN_EPS = 1e-5

_PREC = jax.lax.Precision.HIGHEST

_sc_mesh = plsc.VectorSubcoreMesh(core_axis_name="c", subcore_axis_name="s",
                                  num_cores=2, num_subcores=NSUB)


@functools.partial(
    pl.kernel,
    out_type=jax.ShapeDtypeStruct((N, HH), jnp.float32),
    mesh=_sc_mesh,
    scratch_types=[
        pltpu.VMEM((NCH, CW), jnp.int32),
        pltpu.VMEM((CW, HH), jnp.float32),
        pltpu.VMEM_SHARED((N + NDUM, HH), jnp.float32),
        pltpu.SemaphoreType.DMA,
    ],
)
def _deg_kernel(dstp, ones, zeros, deg_out, dst_v, ones_v, accum, sem):
    c = lax.axis_index("c")
    s = lax.axis_index("s")

    @pl.when(c == 0)
    def _():
        pl.when(s < NCOPY)(
            lambda: pltpu.sync_copy(zeros, accum.at[pl.ds(s * CPR, CPR)]))
        pltpu.sync_copy(dstp.at[s], dst_v)
        pltpu.sync_copy(ones, ones_v)
        plsc.subcore_barrier()

        def chunk(j, carry):
            pltpu.sync_copy(ones_v, accum.at[dst_v.at[j]], add=True)
            return carry

        lax.fori_loop(0, NCH, chunk, 0)
        plsc.subcore_barrier()
        sl = pl.ds(s * CPR, CPR)
        pl.when(s < NCOPY)(
            lambda: pltpu.sync_copy(accum.at[sl], deg_out.at[sl]))


@functools.partial(
    pl.kernel,
    out_type=(jax.ShapeDtypeStruct((N, HH), jnp.float32),
              jax.ShapeDtypeStruct((N, HH), jnp.float32)),
    mesh=_sc_mesh,
    scratch_types=[
        pltpu.VMEM((4, CW), jnp.int32),
        pltpu.VMEM((4, CW), jnp.int32),
        pltpu.VMEM((3, CW, HH), jnp.float32),
        pltpu.VMEM_SHARED((N + NDUM, HH), jnp.float32),
        pltpu.SemaphoreType.DMA,
        pltpu.SemaphoreType.DMA,
    ],
)
def _seg_kernel(h_lo, h_hi, srcp, dstp, zeros, c_lo, c_hi,
                src_v, dst_v, rows_v, accum, sem, sem_i):
    c = lax.axis_index("c")
    s = lax.axis_index("s")
    pl.when(s < NCOPY)(
        lambda: pltpu.sync_copy(zeros, accum.at[pl.ds(s * CPR, CPR)]))
    pltpu.sync_copy(srcp.at[s, 0], src_v.at[0])
    pltpu.sync_copy(dstp.at[s, 0], dst_v.at[0])
    plsc.subcore_barrier()

    def idx_wait(k, slot):
        pltpu.make_async_copy(srcp.at[s, k], src_v.at[slot], sem_i).wait()
        pltpu.make_async_copy(dstp.at[s, k], dst_v.at[slot], sem_i).wait()

    def run(table):
        for k in (1, 2, 3):
            pltpu.async_copy(srcp.at[s, k], src_v.at[k], sem_i)
            pltpu.async_copy(dstp.at[s, k], dst_v.at[k], sem_i)
        pltpu.async_copy(table.at[src_v.at[0]], rows_v.at[0], sem)
        for k in (1, 2):
            idx_wait(k, k)
            pltpu.async_copy(table.at[src_v.at[k]], rows_v.at[k], sem)

        def chunk(j, carry):
            p3 = lax.rem(j, 3)
            p4 = lax.rem(j, 4)
            pltpu.make_async_copy(table.at[src_v.at[p4]],
                                  rows_v.at[p3], sem).wait()
            pltpu.sync_copy(rows_v.at[p3], accum.at[dst_v.at[p4]], add=True)

            @pl.when(j + 3 < NCH)
            def _():
                s4 = lax.rem(j + 3, 4)
                idx_wait(j + 3, s4)
                pltpu.async_copy(table.at[src_v.at[s4]], rows_v.at[p3], sem)

            @pl.when(j + 4 < NCH)
            def _():
                pltpu.async_copy(srcp.at[s, j + 4], src_v.at[p4], sem_i)
                pltpu.async_copy(dstp.at[s, j + 4], dst_v.at[p4], sem_i)
            return carry

        lax.fori_loop(0, NCH, chunk, 0)

    pl.when(c == 0)(lambda: run(h_lo))
    pl.when(c == 1)(lambda: run(h_hi))
    plsc.subcore_barrier()
    sl = pl.ds(s * CPR, CPR)

    @pl.when(s < NCOPY)
    def _():
        pl.when(c == 0)(lambda: pltpu.sync_copy(accum.at[sl], c_lo.at[sl]))
        pl.when(c == 1)(lambda: pltpu.sync_copy(accum.at[sl], c_hi.at[sl]))


@functools.partial(
    pl.kernel,
    out_type=(jax.ShapeDtypeStruct((E, HH), jnp.float32),
              jax.ShapeDtypeStruct((E, HH), jnp.float32)),
    mesh=_sc_mesh,
    scratch_types=[
        pltpu.VMEM((NCH, CW), jnp.int32),
        pltpu.VMEM((4, CW, HH), jnp.float32),
        pltpu.SemaphoreType.DMA,
    ],
)
def _edge_gather_kernel(a_t, b_t, srcp, dstp, sa, sb, idx_v, rows_v, sem):
    c = lax.axis_index("c")
    s = lax.axis_index("s")

    def run(table, idxs, out):
        pltpu.sync_copy(idxs.at[s], idx_v)
        for k in range(3):
            pltpu.async_copy(table.at[idx_v.at[k]], rows_v.at[k], sem)

        def chunk(j, carry):
            p = lax.rem(j, 4)
            pltpu.make_async_copy(table.at[idx_v.at[j]],
                                  rows_v.at[p], sem).wait()

            @pl.when(j + 3 < NCH)
            def _():
                pltpu.async_copy(table.at[idx_v.at[j + 3]],
                                 rows_v.at[lax.rem(j + 3, 4)], sem)
            pltpu.sync_copy(rows_v.at[p], out.at[pl.ds(s * ES + j * CW, CW)])
            return carry

        lax.fori_loop(0, NCH - 1, chunk, 0)
        p_last = (NCH - 1) % 4
        pltpu.make_async_copy(table.at[idx_v.at[NCH - 1]],
                              rows_v.at[p_last], sem).wait()
        pltpu.sync_copy(rows_v.at[p_last].at[pl.ds(0, TAIL)],
                        out.at[pl.ds(s * ES + (NCH - 1) * CW, TAIL)])

    pl.when(c == 0)(lambda: run(a_t, srcp, sa))
    pl.when(c == 1)(lambda: run(b_t, dstp, sb))


def _embed_body(x_ref, wlo_ref, whi_ref, blo_ref, bhi_ref, olo_ref, ohi_ref):
    x = x_ref[...]
    olo_ref[...] = jnp.dot(x, wlo_ref[...],
                           preferred_element_type=jnp.float32, precision=jax.lax.Precision.HIGHEST) + blo_ref[...]
    ohi_ref[...] = jnp.dot(x, whi_ref[...],
                           preferred_element_type=jnp.float32, precision=jax.lax.Precision.HIGHEST) + bhi_ref[...]


def _embed(x, W, b):
    R = 2000
    return pl.pallas_call(
        _embed_body,
        grid=(N // R,),
        in_specs=[
            pl.BlockSpec((R, DIN), lambda i: (i, 0)),
            pl.BlockSpec((DIN, HH), lambda i: (0, 0)),
            pl.BlockSpec((DIN, HH), lambda i: (0, 0)),
            pl.BlockSpec((1, HH), lambda i: (0, 0)),
            pl.BlockSpec((1, HH), lambda i: (0, 0)),
        ],
        out_specs=(pl.BlockSpec((R, HH), lambda i: (i, 0)),
                   pl.BlockSpec((R, HH), lambda i: (i, 0))),
        out_shape=(jax.ShapeDtypeStruct((N, HH), jnp.float32),
                   jax.ShapeDtypeStruct((N, HH), jnp.float32)),
    )(x, W[:, :HH], W[:, HH:], b[:HH].reshape(1, HH), b[HH:].reshape(1, HH))


def _layer_body(hlo_ref, hhi_ref, clo_ref, chi_ref, deg_ref,
                wtlo_ref, wthi_ref, wblo_ref, wbhi_ref,
                b_ref, bnsc_ref, bnsh_ref, olo_ref, ohi_ref):
    r = 1.0 / jnp.maximum(deg_ref[...][:, 0:1], 1.0)
    f32 = jnp.float32
    bundle = (jnp.dot(hlo_ref[...], wtlo_ref[...], preferred_element_type=f32, precision=_PREC)
              + jnp.dot(hhi_ref[...], wthi_ref[...], preferred_element_type=f32, precision=_PREC)
              + jnp.dot(clo_ref[...] * r, wblo_ref[...], preferred_element_type=f32, precision=_PREC)
              + jnp.dot(chi_ref[...] * r, wbhi_ref[...], preferred_element_type=f32, precision=_PREC)
              + b_ref[...])
    inv = 1.0 / jnp.maximum(
        jnp.sqrt(jnp.sum(bundle * bundle, axis=1, keepdims=True)), 1e-12)
    t = jnp.maximum(bundle * inv, 0.0) * bnsc_ref[...] + bnsh_ref[...]
    olo_ref[...] = hlo_ref[...] + t[:, :HH]
    ohi_ref[...] = hhi_ref[...] + t[:, HH:]


def _layer(h_lo, h_hi, c_lo, c_hi, deg16, Wt, Wb, b, bnsc, bnsh):
    R = 2000
    full = lambda i: (0, 0)
    blk = lambda i: (i, 0)
    return pl.pallas_call(
        _layer_body,
        grid=(N // R,),
        in_specs=[
            pl.BlockSpec((R, HH), blk), pl.BlockSpec((R, HH), blk),
            pl.BlockSpec((R, HH), blk), pl.BlockSpec((R, HH), blk),
            pl.BlockSpec((R, HH), blk),
            pl.BlockSpec((HH, H), full), pl.BlockSpec((HH, H), full),
            pl.BlockSpec((HH, H), full), pl.BlockSpec((HH, H), full),
            pl.BlockSpec((1, H), full), pl.BlockSpec((1, H), full),
            pl.BlockSpec((1, H), full),
        ],
        out_specs=(pl.BlockSpec((R, HH), blk), pl.BlockSpec((R, HH), blk)),
        out_shape=(jax.ShapeDtypeStruct((N, HH), jnp.float32),
                   jax.ShapeDtypeStruct((N, HH), jnp.float32)),
    )(h_lo, h_hi, c_lo, c_hi, deg16,
      Wt[:HH], Wt[HH:], Wb[:HH], Wb[HH:],
      b.reshape(1, H), bnsc.reshape(1, H), bnsh.reshape(1, H))


def _pack_bf16(v):
    vb = v.astype(jnp.bfloat16)
    lo = lax.bitcast_convert_type(vb[:, :HH], jnp.uint16).astype(jnp.uint32)
    hi = lax.bitcast_convert_type(vb[:, HH:], jnp.uint16).astype(jnp.uint32)
    return lax.bitcast_convert_type(lo | (hi << 16), jnp.float32)


def _unpack_bf16(w):
    u = lax.bitcast_convert_type(w, jnp.uint32)
    lo = lax.bitcast_convert_type((u & 0xFFFF).astype(jnp.uint16),
                                  jnp.bfloat16).astype(jnp.float32)
    hi = lax.bitcast_convert_type((u >> 16).astype(jnp.uint16),
                                  jnp.bfloat16).astype(jnp.float32)
    return lo, hi


def _ab_body(hlo_ref, hhi_ref, walo_ref, wahi_ref, wblo_ref, wbhi_ref,
             a_ref, b_ref):
    f32 = jnp.float32
    hlo = hlo_ref[...]
    hhi = hhi_ref[...]
    a = (jnp.dot(hlo, walo_ref[...], preferred_element_type=f32, precision=_PREC)
         + jnp.dot(hhi, wahi_ref[...], preferred_element_type=f32, precision=_PREC))
    b = (jnp.dot(hlo, wblo_ref[...], preferred_element_type=f32, precision=_PREC)
         + jnp.dot(hhi, wbhi_ref[...], preferred_element_type=f32, precision=_PREC))
    a_ref[...] = _pack_bf16(a)
    b_ref[...] = _pack_bf16(b)


def _ab(h_lo, h_hi, Wa, Wb):
    R = 2000
    full = lambda i: (0, 0)
    blk = lambda i: (i, 0)
    return pl.pallas_call(
        _ab_body,
        grid=(N // R,),
        in_specs=[
            pl.BlockSpec((R, HH), blk), pl.BlockSpec((R, HH), blk),
            pl.BlockSpec((HH, H), full), pl.BlockSpec((HH, H), full),
            pl.BlockSpec((HH, H), full), pl.BlockSpec((HH, H), full),
        ],
        out_specs=(pl.BlockSpec((R, HH), blk), pl.BlockSpec((R, HH), blk)),
        out_shape=(jax.ShapeDtypeStruct((N, HH), jnp.float32),
                   jax.ShapeDtypeStruct((N, HH), jnp.float32)),
    )(h_lo, h_hi, Wa[:HH], Wa[HH:], Wb[:HH], Wb[HH:])


def _mlp_body(sa_ref, sb_ref, b0lo_ref, b0hi_ref,
              w1lo_ref, w1hi_ref, b1_ref, w2_ref, b2_ref, o_ref):
    f32 = jnp.float32
    sa_lo, sa_hi = _unpack_bf16(sa_ref[...])
    sb_lo, sb_hi = _unpack_bf16(sb_ref[...])
    y0_lo = jnp.maximum(sa_lo + sb_lo + b0lo_ref[...], 0.0)
    y0_hi = jnp.maximum(sa_hi + sb_hi + b0hi_ref[...], 0.0)
    y1 = jnp.maximum(
        jnp.dot(y0_lo, w1lo_ref[...], preferred_element_type=f32, precision=_PREC)
        + jnp.dot(y0_hi, w1hi_ref[...], preferred_element_type=f32, precision=_PREC)
        + b1_ref[...], 0.0)
    o_ref[...] = jnp.dot(y1, w2_ref[...], preferred_element_type=f32, precision=_PREC) + b2_ref[...]


def _mlp(sa, sb, b0, W1, b1, W2, b2):
    R = 2000
    full = lambda i: (0, 0)
    blk = lambda i: (i, 0)
    return pl.pallas_call(
        _mlp_body,
        grid=(E // R,),
        in_specs=[
            pl.BlockSpec((R, HH), blk), pl.BlockSpec((R, HH), blk),
            pl.BlockSpec((1, HH), full), pl.BlockSpec((1, HH), full),
            pl.BlockSpec((HH, HH), full), pl.BlockSpec((HH, HH), full),
            pl.BlockSpec((1, HH), full),
            pl.BlockSpec((HH, 2), full), pl.BlockSpec((1, 2), full),
        ],
        out_specs=pl.BlockSpec((R, 2), blk),
        out_shape=jax.ShapeDtypeStruct((E, 2), jnp.float32),
    )(sa, sb, b0[:HH].reshape(1, HH), b0[HH:].reshape(1, HH),
      W1[:HH], W1[HH:], b1.reshape(1, HH), W2, b2.reshape(1, 2))


def kernel(x, edge_index, W_embed, b_embed,
           W_sage0, b_sage0, bn_g0, bn_b0, bn_m0, bn_v0,
           W_sage1, b_sage1, bn_g1, bn_b1, bn_m1, bn_v1,
           W_sage2, b_sage2, bn_g2, bn_b2, bn_m2, bn_v2,
           W_mlp0, b_mlp0, W_mlp1, b_mlp1, W_mlp2, b_mlp2):
    src = edge_index[0]
    dst = edge_index[1]
    srcp = jnp.pad(src.reshape(NSUB, ES),
                   ((0, 0), (0, PADE - ES))).reshape(NSUB, NCH, CW)
    dstp = jnp.pad(dst.reshape(NSUB, ES), ((0, 0), (0, PADE - ES)),
                   constant_values=N).reshape(NSUB, NCH, CW)
    zeros = jnp.zeros((CPR, HH), jnp.float32)
    ones_cw = jnp.ones((CW, HH), jnp.float32)

    deg16 = _deg_kernel(dstp, ones_cw, zeros)
    h_lo, h_hi = _embed(x, W_embed, b_embed)

    sages = ((W_sage0, b_sage0, bn_g0, bn_b0, bn_m0, bn_v0),
             (W_sage1, b_sage1, bn_g1, bn_b1, bn_m1, bn_v1),
             (W_sage2, b_sage2, bn_g2, bn_b2, bn_m2, bn_v2))
    for W, b, g, bb, m, v in sages:
        c_lo, c_hi = _seg_kernel(h_lo, h_hi, srcp, dstp, zeros)
        bnsc = g * jax.lax.rsqrt(v + BN_EPS)
        bnsh = bb - m * bnsc
        h_lo, h_hi = _layer(h_lo, h_hi, c_lo, c_hi, deg16,
                            W[:H], W[H:], b, bnsc, bnsh)

    a_t, b_t = _ab(h_lo, h_hi, W_mlp0[:H], W_mlp0[H:])
    sa, sb = _edge_gather_kernel(a_t, b_t, srcp, dstp)
    return _mlp(sa, sb, b_mlp0, W_mlp1, b_mlp1, W_mlp2, b_mlp2)


# speedup vs baseline: 4.3289x; 1.1969x over previous
import functools

import jax
import jax.numpy as jnp
from jax import lax
from jax.experimental import pallas as pl
from jax.experimental.pallas import tpu as pltpu
from jax.experimental.pallas import tpu_sc as plsc

N = 10000
E = 160000
DIN = 1024
H = 256
HH = 128
NSUB = 16
ES = E // NSUB
CW = 128
NCH = (ES + CW - 1) // CW
TAIL = ES - (NCH - 1) * CW
PADE = NCH * CW
NDUM = 8
NCOPY = 10
CPR = N // NCOPY
BN_EPS = 1e-5

_PREC = jax.lax.Precision.HIGHEST

_sc_mesh = plsc.VectorSubcoreMesh(core_axis_name="c", subcore_axis_name="s",
                                  num_cores=2, num_subcores=NSUB)


@functools.partial(
    pl.kernel,
    out_type=jax.ShapeDtypeStruct((N, HH), jnp.float32),
    mesh=_sc_mesh,
    scratch_types=[
        pltpu.VMEM((NCH, CW), jnp.int32),
        pltpu.VMEM((CW, HH), jnp.float32),
        pltpu.VMEM_SHARED((N + NDUM, HH), jnp.float32),
        pltpu.SemaphoreType.DMA,
    ],
)
def _deg_kernel(dstp, ones, zeros, deg_out, dst_v, ones_v, accum, sem):
    c = lax.axis_index("c")
    s = lax.axis_index("s")

    @pl.when(c == 0)
    def _():
        pl.when(s < NCOPY)(
            lambda: pltpu.sync_copy(zeros, accum.at[pl.ds(s * CPR, CPR)]))
        pltpu.sync_copy(dstp.at[s], dst_v)
        pltpu.sync_copy(ones, ones_v)
        plsc.subcore_barrier()

        def chunk(j, carry):
            pltpu.sync_copy(ones_v, accum.at[dst_v.at[j]], add=True)
            return carry

        lax.fori_loop(0, NCH, chunk, 0)
        plsc.subcore_barrier()
        sl = pl.ds(s * CPR, CPR)
        pl.when(s < NCOPY)(
            lambda: pltpu.sync_copy(accum.at[sl], deg_out.at[sl]))


@functools.partial(
    pl.kernel,
    out_type=(jax.ShapeDtypeStruct((N, HH), jnp.float32),
              jax.ShapeDtypeStruct((N, HH), jnp.float32)),
    mesh=_sc_mesh,
    scratch_types=[
        pltpu.VMEM((4, CW), jnp.int32),
        pltpu.VMEM((4, CW), jnp.int32),
        pltpu.VMEM((3, CW, HH), jnp.float32),
        pltpu.VMEM_SHARED((N + NDUM, HH), jnp.float32),
        pltpu.SemaphoreType.DMA,
        pltpu.SemaphoreType.DMA,
    ],
)
def _seg_kernel(h_lo, h_hi, srcp, dstp, zeros, c_lo, c_hi,
                src_v, dst_v, rows_v, accum, sem, sem_i):
    c = lax.axis_index("c")
    s = lax.axis_index("s")
    pl.when(s < NCOPY)(
        lambda: pltpu.sync_copy(zeros, accum.at[pl.ds(s * CPR, CPR)]))
    pltpu.sync_copy(srcp.at[s, 0], src_v.at[0])
    pltpu.sync_copy(dstp.at[s, 0], dst_v.at[0])
    plsc.subcore_barrier()

    def idx_wait(k, slot):
        pltpu.make_async_copy(srcp.at[s, k], src_v.at[slot], sem_i).wait()
        pltpu.make_async_copy(dstp.at[s, k], dst_v.at[slot], sem_i).wait()

    def run(table):
        for k in (1, 2, 3):
            pltpu.async_copy(srcp.at[s, k], src_v.at[k], sem_i)
            pltpu.async_copy(dstp.at[s, k], dst_v.at[k], sem_i)
        pltpu.async_copy(table.at[src_v.at[0]], rows_v.at[0], sem)
        for k in (1, 2):
            idx_wait(k, k)
            pltpu.async_copy(table.at[src_v.at[k]], rows_v.at[k], sem)

        def chunk(j, carry):
            p3 = lax.rem(j, 3)
            p4 = lax.rem(j, 4)
            pltpu.make_async_copy(table.at[src_v.at[p4]],
                                  rows_v.at[p3], sem).wait()
            pltpu.sync_copy(rows_v.at[p3], accum.at[dst_v.at[p4]], add=True)

            @pl.when(j + 3 < NCH)
            def _():
                s4 = lax.rem(j + 3, 4)
                idx_wait(j + 3, s4)
                pltpu.async_copy(table.at[src_v.at[s4]], rows_v.at[p3], sem)

            @pl.when(j + 4 < NCH)
            def _():
                pltpu.async_copy(srcp.at[s, j + 4], src_v.at[p4], sem_i)
                pltpu.async_copy(dstp.at[s, j + 4], dst_v.at[p4], sem_i)
            return carry

        lax.fori_loop(0, NCH, chunk, 0)

    pl.when(c == 0)(lambda: run(h_lo))
    pl.when(c == 1)(lambda: run(h_hi))
    plsc.subcore_barrier()
    sl = pl.ds(s * CPR, CPR)

    @pl.when(s < NCOPY)
    def _():
        pl.when(c == 0)(lambda: pltpu.sync_copy(accum.at[sl], c_lo.at[sl]))
        pl.when(c == 1)(lambda: pltpu.sync_copy(accum.at[sl], c_hi.at[sl]))


@functools.partial(
    pl.kernel,
    out_type=(jax.ShapeDtypeStruct((E, HH), jnp.float32),
              jax.ShapeDtypeStruct((E, HH), jnp.float32)),
    mesh=_sc_mesh,
    scratch_types=[
        pltpu.VMEM((NCH, CW), jnp.int32),
        pltpu.VMEM((4, CW, HH), jnp.float32),
        pltpu.SemaphoreType.DMA,
    ],
)
def _edge_gather_kernel(a_t, b_t, srcp, dstp, sa, sb, idx_v, rows_v, sem):
    c = lax.axis_index("c")
    s = lax.axis_index("s")

    def run(table, idxs, out):
        pltpu.sync_copy(idxs.at[s], idx_v)
        for k in range(3):
            pltpu.async_copy(table.at[idx_v.at[k]], rows_v.at[k], sem)

        def chunk(j, carry):
            p = lax.rem(j, 4)
            pltpu.make_async_copy(table.at[idx_v.at[j]],
                                  rows_v.at[p], sem).wait()

            @pl.when(j + 3 < NCH)
            def _():
                pltpu.async_copy(table.at[idx_v.at[j + 3]],
                                 rows_v.at[lax.rem(j + 3, 4)], sem)
            pltpu.sync_copy(rows_v.at[p], out.at[pl.ds(s * ES + j * CW, CW)])
            return carry

        lax.fori_loop(0, NCH - 1, chunk, 0)
        p_last = (NCH - 1) % 4
        pltpu.make_async_copy(table.at[idx_v.at[NCH - 1]],
                              rows_v.at[p_last], sem).wait()
        pltpu.sync_copy(rows_v.at[p_last].at[pl.ds(0, TAIL)],
                        out.at[pl.ds(s * ES + (NCH - 1) * CW, TAIL)])

    pl.when(c == 0)(lambda: run(a_t, srcp, sa))
    pl.when(c == 1)(lambda: run(b_t, dstp, sb))


def _embed_body(x_ref, wlo_ref, whi_ref, blo_ref, bhi_ref, olo_ref, ohi_ref):
    x = x_ref[...]
    olo_ref[...] = jnp.dot(x, wlo_ref[...],
                           preferred_element_type=jnp.float32, precision=_PREC) + blo_ref[...]
    ohi_ref[...] = jnp.dot(x, whi_ref[...],
                           preferred_element_type=jnp.float32, precision=_PREC) + bhi_ref[...]


def _embed(x, W, b):
    R = 2000
    return pl.pallas_call(
        _embed_body,
        grid=(N // R,),
        in_specs=[
            pl.BlockSpec((R, DIN), lambda i: (i, 0)),
            pl.BlockSpec((DIN, HH), lambda i: (0, 0)),
            pl.BlockSpec((DIN, HH), lambda i: (0, 0)),
            pl.BlockSpec((1, HH), lambda i: (0, 0)),
            pl.BlockSpec((1, HH), lambda i: (0, 0)),
        ],
        out_specs=(pl.BlockSpec((R, HH), lambda i: (i, 0)),
                   pl.BlockSpec((R, HH), lambda i: (i, 0))),
        out_shape=(jax.ShapeDtypeStruct((N, HH), jnp.float32),
                   jax.ShapeDtypeStruct((N, HH), jnp.float32)),
    )(x, W[:, :HH], W[:, HH:], b[:HH].reshape(1, HH), b[HH:].reshape(1, HH))


def _layer_body(hlo_ref, hhi_ref, clo_ref, chi_ref, deg_ref,
                wtlo_ref, wthi_ref, wblo_ref, wbhi_ref,
                b_ref, bnsc_ref, bnsh_ref, olo_ref, ohi_ref):
    r = 1.0 / jnp.maximum(deg_ref[...][:, 0:1], 1.0)
    f32 = jnp.float32
    bundle = (jnp.dot(hlo_ref[...], wtlo_ref[...], preferred_element_type=f32, precision=_PREC)
              + jnp.dot(hhi_ref[...], wthi_ref[...], preferred_element_type=f32, precision=_PREC)
              + jnp.dot(clo_ref[...] * r, wblo_ref[...], preferred_element_type=f32, precision=_PREC)
              + jnp.dot(chi_ref[...] * r, wbhi_ref[...], preferred_element_type=f32, precision=_PREC)
              + b_ref[...])
    inv = 1.0 / jnp.maximum(
        jnp.sqrt(jnp.sum(bundle * bundle, axis=1, keepdims=True)), 1e-12)
    t = jnp.maximum(bundle * inv, 0.0) * bnsc_ref[...] + bnsh_ref[...]
    olo_ref[...] = hlo_ref[...] + t[:, :HH]
    ohi_ref[...] = hhi_ref[...] + t[:, HH:]


def _layer(h_lo, h_hi, c_lo, c_hi, deg16, Wt, Wb, b, bnsc, bnsh):
    R = 2000
    full = lambda i: (0, 0)
    blk = lambda i: (i, 0)
    return pl.pallas_call(
        _layer_body,
        grid=(N // R,),
        in_specs=[
            pl.BlockSpec((R, HH), blk), pl.BlockSpec((R, HH), blk),
            pl.BlockSpec((R, HH), blk), pl.BlockSpec((R, HH), blk),
            pl.BlockSpec((R, HH), blk),
            pl.BlockSpec((HH, H), full), pl.BlockSpec((HH, H), full),
            pl.BlockSpec((HH, H), full), pl.BlockSpec((HH, H), full),
            pl.BlockSpec((1, H), full), pl.BlockSpec((1, H), full),
            pl.BlockSpec((1, H), full),
        ],
        out_specs=(pl.BlockSpec((R, HH), blk), pl.BlockSpec((R, HH), blk)),
        out_shape=(jax.ShapeDtypeStruct((N, HH), jnp.float32),
                   jax.ShapeDtypeStruct((N, HH), jnp.float32)),
    )(h_lo, h_hi, c_lo, c_hi, deg16,
      Wt[:HH], Wt[HH:], Wb[:HH], Wb[HH:],
      b.reshape(1, H), bnsc.reshape(1, H), bnsh.reshape(1, H))


def _pack_bf16(v):
    vb = v.astype(jnp.bfloat16)
    lo = lax.bitcast_convert_type(vb[:, :HH], jnp.uint16).astype(jnp.uint32)
    hi = lax.bitcast_convert_type(vb[:, HH:], jnp.uint16).astype(jnp.uint32)
    return lax.bitcast_convert_type(lo | (hi << 16), jnp.float32)


def _unpack_bf16(w):
    u = lax.bitcast_convert_type(w, jnp.uint32)
    lo = lax.bitcast_convert_type((u & 0xFFFF).astype(jnp.uint16),
                                  jnp.bfloat16)
    hi = lax.bitcast_convert_type((u >> 16).astype(jnp.uint16),
                                  jnp.bfloat16)
    return lo, hi


def _ab_body(hlo_ref, hhi_ref, walo_ref, wahi_ref, wblo_ref, wbhi_ref,
             a_ref, b_ref):
    f32 = jnp.float32
    hlo = hlo_ref[...]
    hhi = hhi_ref[...]
    a = (jnp.dot(hlo, walo_ref[...], preferred_element_type=f32, precision=_PREC)
         + jnp.dot(hhi, wahi_ref[...], preferred_element_type=f32, precision=_PREC))
    b = (jnp.dot(hlo, wblo_ref[...], preferred_element_type=f32, precision=_PREC)
         + jnp.dot(hhi, wbhi_ref[...], preferred_element_type=f32, precision=_PREC))
    a_ref[...] = _pack_bf16(a)
    b_ref[...] = _pack_bf16(b)


def _ab(h_lo, h_hi, Wa, Wb):
    R = 2000
    full = lambda i: (0, 0)
    blk = lambda i: (i, 0)
    return pl.pallas_call(
        _ab_body,
        grid=(N // R,),
        in_specs=[
            pl.BlockSpec((R, HH), blk), pl.BlockSpec((R, HH), blk),
            pl.BlockSpec((HH, H), full), pl.BlockSpec((HH, H), full),
            pl.BlockSpec((HH, H), full), pl.BlockSpec((HH, H), full),
        ],
        out_specs=(pl.BlockSpec((R, HH), blk), pl.BlockSpec((R, HH), blk)),
        out_shape=(jax.ShapeDtypeStruct((N, HH), jnp.float32),
                   jax.ShapeDtypeStruct((N, HH), jnp.float32)),
    )(h_lo, h_hi, Wa[:HH], Wa[HH:], Wb[:HH], Wb[HH:])


def _mlp_body(sa_ref, sb_ref, b0lo_ref, b0hi_ref,
              w1lo_ref, w1hi_ref, b1_ref, w2_ref, b2_ref, o_ref):
    f32 = jnp.float32
    sa_lo, sa_hi = _unpack_bf16(sa_ref[...])
    sb_lo, sb_hi = _unpack_bf16(sb_ref[...])
    zero = jnp.bfloat16(0)
    y0_lo = jnp.maximum(sa_lo + sb_lo + b0lo_ref[...], zero)
    y0_hi = jnp.maximum(sa_hi + sb_hi + b0hi_ref[...], zero)
    y1 = jnp.maximum(
        jnp.dot(y0_lo, w1lo_ref[...], preferred_element_type=f32)
        + jnp.dot(y0_hi, w1hi_ref[...], preferred_element_type=f32)
        + b1_ref[...], 0.0)
    o_ref[...] = jnp.dot(y1, w2_ref[...], preferred_element_type=f32,
                         precision=_PREC) + b2_ref[...]


def _mlp(sa, sb, b0, W1, b1, W2, b2):
    R = 4000
    full = lambda i: (0, 0)
    blk = lambda i: (i, 0)
    bf = jnp.bfloat16
    return pl.pallas_call(
        _mlp_body,
        grid=(E // R,),
        in_specs=[
            pl.BlockSpec((R, HH), blk), pl.BlockSpec((R, HH), blk),
            pl.BlockSpec((1, HH), full), pl.BlockSpec((1, HH), full),
            pl.BlockSpec((HH, HH), full), pl.BlockSpec((HH, HH), full),
            pl.BlockSpec((1, HH), full),
            pl.BlockSpec((HH, 2), full), pl.BlockSpec((1, 2), full),
        ],
        out_specs=pl.BlockSpec((R, 2), blk),
        out_shape=jax.ShapeDtypeStruct((E, 2), jnp.float32),
    )(sa, sb, b0[:HH].reshape(1, HH).astype(bf), b0[HH:].reshape(1, HH).astype(bf),
      W1[:HH].astype(bf), W1[HH:].astype(bf), b1.reshape(1, HH),
      W2, b2.reshape(1, 2))


def kernel(x, edge_index, W_embed, b_embed,
           W_sage0, b_sage0, bn_g0, bn_b0, bn_m0, bn_v0,
           W_sage1, b_sage1, bn_g1, bn_b1, bn_m1, bn_v1,
           W_sage2, b_sage2, bn_g2, bn_b2, bn_m2, bn_v2,
           W_mlp0, b_mlp0, W_mlp1, b_mlp1, W_mlp2, b_mlp2):
    src = edge_index[0]
    dst = edge_index[1]
    srcp = jnp.pad(src.reshape(NSUB, ES),
                   ((0, 0), (0, PADE - ES))).reshape(NSUB, NCH, CW)
    dstp = jnp.pad(dst.reshape(NSUB, ES), ((0, 0), (0, PADE - ES)),
                   constant_values=N).reshape(NSUB, NCH, CW)
    zeros = jnp.zeros((CPR, HH), jnp.float32)
    ones_cw = jnp.ones((CW, HH), jnp.float32)

    deg16 = _deg_kernel(dstp, ones_cw, zeros)
    h_lo, h_hi = _embed(x, W_embed, b_embed)

    sages = ((W_sage0, b_sage0, bn_g0, bn_b0, bn_m0, bn_v0),
             (W_sage1, b_sage1, bn_g1, bn_b1, bn_m1, bn_v1),
             (W_sage2, b_sage2, bn_g2, bn_b2, bn_m2, bn_v2))
    for W, b, g, bb, m, v in sages:
        c_lo, c_hi = _seg_kernel(h_lo, h_hi, srcp, dstp, zeros)
        bnsc = g * jax.lax.rsqrt(v + BN_EPS)
        bnsh = bb - m * bnsc
        h_lo, h_hi = _layer(h_lo, h_hi, c_lo, c_hi, deg16,
                            W[:H], W[H:], b, bnsc, bnsh)

    a_t, b_t = _ab(h_lo, h_hi, W_mlp0[:H], W_mlp0[H:])
    sa, sb = _edge_gather_kernel(a_t, b_t, srcp, dstp)
    return _mlp(sa, sb, b_mlp0, W_mlp1, b_mlp1, W_mlp2, b_mlp2)


# speedup vs baseline: 4.9687x; 1.1478x over previous
import functools

import jax
import jax.numpy as jnp
from jax import lax
from jax.experimental import pallas as pl
from jax.experimental.pallas import tpu as pltpu
from jax.experimental.pallas import tpu_sc as plsc

N = 10000
E = 160000
DIN = 1024
H = 256
HH = 128
NSUB = 16
ES = E // NSUB
CW = 128
NCH = (ES + CW - 1) // CW
TAIL = ES - (NCH - 1) * CW
PADE = NCH * CW
NDUM = 8
NCOPY = 10
CPR = N // NCOPY
BN_EPS = 1e-5

_PREC = jax.lax.Precision.DEFAULT

_sc_mesh = plsc.VectorSubcoreMesh(core_axis_name="c", subcore_axis_name="s",
                                  num_cores=2, num_subcores=NSUB)


@functools.partial(
    pl.kernel,
    out_type=jax.ShapeDtypeStruct((N, HH), jnp.float32),
    mesh=_sc_mesh,
    scratch_types=[
        pltpu.VMEM((NCH, CW), jnp.int32),
        pltpu.VMEM((CW, HH), jnp.float32),
        pltpu.VMEM_SHARED((N + NDUM, HH), jnp.float32),
        pltpu.SemaphoreType.DMA,
    ],
)
def _deg_kernel(dstp, ones, zeros, deg_out, dst_v, ones_v, accum, sem):
    c = lax.axis_index("c")
    s = lax.axis_index("s")

    @pl.when(c == 0)
    def _():
        pl.when(s < NCOPY)(
            lambda: pltpu.sync_copy(zeros, accum.at[pl.ds(s * CPR, CPR)]))
        pltpu.sync_copy(dstp.at[s], dst_v)
        pltpu.sync_copy(ones, ones_v)
        plsc.subcore_barrier()

        def chunk(j, carry):
            pltpu.sync_copy(ones_v, accum.at[dst_v.at[j]], add=True)
            return carry

        lax.fori_loop(0, NCH, chunk, 0)
        plsc.subcore_barrier()
        sl = pl.ds(s * CPR, CPR)
        pl.when(s < NCOPY)(
            lambda: pltpu.sync_copy(accum.at[sl], deg_out.at[sl]))


@functools.partial(
    pl.kernel,
    out_type=(jax.ShapeDtypeStruct((N, HH), jnp.float32),
              jax.ShapeDtypeStruct((N, HH), jnp.float32)),
    mesh=_sc_mesh,
    scratch_types=[
        pltpu.VMEM((4, CW), jnp.int32),
        pltpu.VMEM((4, CW), jnp.int32),
        pltpu.VMEM((3, CW, HH), jnp.float32),
        pltpu.VMEM_SHARED((N + NDUM, HH), jnp.float32),
        pltpu.SemaphoreType.DMA,
        pltpu.SemaphoreType.DMA,
    ],
)
def _seg_kernel(h_lo, h_hi, srcp, dstp, zeros, c_lo, c_hi,
                src_v, dst_v, rows_v, accum, sem, sem_i):
    c = lax.axis_index("c")
    s = lax.axis_index("s")
    pl.when(s < NCOPY)(
        lambda: pltpu.sync_copy(zeros, accum.at[pl.ds(s * CPR, CPR)]))
    pltpu.sync_copy(srcp.at[s, 0], src_v.at[0])
    pltpu.sync_copy(dstp.at[s, 0], dst_v.at[0])
    plsc.subcore_barrier()

    def idx_wait(k, slot):
        pltpu.make_async_copy(srcp.at[s, k], src_v.at[slot], sem_i).wait()
        pltpu.make_async_copy(dstp.at[s, k], dst_v.at[slot], sem_i).wait()

    def run(table):
        for k in (1, 2, 3):
            pltpu.async_copy(srcp.at[s, k], src_v.at[k], sem_i)
            pltpu.async_copy(dstp.at[s, k], dst_v.at[k], sem_i)
        pltpu.async_copy(table.at[src_v.at[0]], rows_v.at[0], sem)
        for k in (1, 2):
            idx_wait(k, k)
            pltpu.async_copy(table.at[src_v.at[k]], rows_v.at[k], sem)

        def chunk(j, carry):
            p3 = lax.rem(j, 3)
            p4 = lax.rem(j, 4)
            pltpu.make_async_copy(table.at[src_v.at[p4]],
                                  rows_v.at[p3], sem).wait()
            pltpu.sync_copy(rows_v.at[p3], accum.at[dst_v.at[p4]], add=True)

            @pl.when(j + 3 < NCH)
            def _():
                s4 = lax.rem(j + 3, 4)
                idx_wait(j + 3, s4)
                pltpu.async_copy(table.at[src_v.at[s4]], rows_v.at[p3], sem)

            @pl.when(j + 4 < NCH)
            def _():
                pltpu.async_copy(srcp.at[s, j + 4], src_v.at[p4], sem_i)
                pltpu.async_copy(dstp.at[s, j + 4], dst_v.at[p4], sem_i)
            return carry

        lax.fori_loop(0, NCH, chunk, 0)

    pl.when(c == 0)(lambda: run(h_lo))
    pl.when(c == 1)(lambda: run(h_hi))
    plsc.subcore_barrier()
    sl = pl.ds(s * CPR, CPR)

    @pl.when(s < NCOPY)
    def _():
        pl.when(c == 0)(lambda: pltpu.sync_copy(accum.at[sl], c_lo.at[sl]))
        pl.when(c == 1)(lambda: pltpu.sync_copy(accum.at[sl], c_hi.at[sl]))


@functools.partial(
    pl.kernel,
    out_type=(jax.ShapeDtypeStruct((E, HH), jnp.float32),
              jax.ShapeDtypeStruct((E, HH), jnp.float32)),
    mesh=_sc_mesh,
    scratch_types=[
        pltpu.VMEM((NCH, CW), jnp.int32),
        pltpu.VMEM((4, CW, HH), jnp.float32),
        pltpu.SemaphoreType.DMA,
    ],
)
def _edge_gather_kernel(a_t, b_t, srcp, dstp, sa, sb, idx_v, rows_v, sem):
    c = lax.axis_index("c")
    s = lax.axis_index("s")

    def run(table, idxs, out):
        pltpu.sync_copy(idxs.at[s], idx_v)
        for k in range(3):
            pltpu.async_copy(table.at[idx_v.at[k]], rows_v.at[k], sem)

        def chunk(j, carry):
            p = lax.rem(j, 4)
            pltpu.make_async_copy(table.at[idx_v.at[j]],
                                  rows_v.at[p], sem).wait()

            @pl.when(j + 3 < NCH)
            def _():
                pltpu.async_copy(table.at[idx_v.at[j + 3]],
                                 rows_v.at[lax.rem(j + 3, 4)], sem)
            pltpu.sync_copy(rows_v.at[p], out.at[pl.ds(s * ES + j * CW, CW)])
            return carry

        lax.fori_loop(0, NCH - 1, chunk, 0)
        p_last = (NCH - 1) % 4
        pltpu.make_async_copy(table.at[idx_v.at[NCH - 1]],
                              rows_v.at[p_last], sem).wait()
        pltpu.sync_copy(rows_v.at[p_last].at[pl.ds(0, TAIL)],
                        out.at[pl.ds(s * ES + (NCH - 1) * CW, TAIL)])

    pl.when(c == 0)(lambda: run(a_t, srcp, sa))
    pl.when(c == 1)(lambda: run(b_t, dstp, sb))


def _embed_body(x_ref, wlo_ref, whi_ref, blo_ref, bhi_ref, olo_ref, ohi_ref):
    x = x_ref[...]
    olo_ref[...] = jnp.dot(x, wlo_ref[...],
                           preferred_element_type=jnp.float32, precision=_PREC) + blo_ref[...]
    ohi_ref[...] = jnp.dot(x, whi_ref[...],
                           preferred_element_type=jnp.float32, precision=_PREC) + bhi_ref[...]


def _embed(x, W, b):
    R = 2000
    return pl.pallas_call(
        _embed_body,
        grid=(N // R,),
        in_specs=[
            pl.BlockSpec((R, DIN), lambda i: (i, 0)),
            pl.BlockSpec((DIN, HH), lambda i: (0, 0)),
            pl.BlockSpec((DIN, HH), lambda i: (0, 0)),
            pl.BlockSpec((1, HH), lambda i: (0, 0)),
            pl.BlockSpec((1, HH), lambda i: (0, 0)),
        ],
        out_specs=(pl.BlockSpec((R, HH), lambda i: (i, 0)),
                   pl.BlockSpec((R, HH), lambda i: (i, 0))),
        out_shape=(jax.ShapeDtypeStruct((N, HH), jnp.float32),
                   jax.ShapeDtypeStruct((N, HH), jnp.float32)),
    )(x, W[:, :HH], W[:, HH:], b[:HH].reshape(1, HH), b[HH:].reshape(1, HH))


def _layer_body(hlo_ref, hhi_ref, clo_ref, chi_ref, deg_ref,
                wtlo_ref, wthi_ref, wblo_ref, wbhi_ref,
                b_ref, bnsc_ref, bnsh_ref, olo_ref, ohi_ref):
    r = 1.0 / jnp.maximum(deg_ref[...][:, 0:1], 1.0)
    f32 = jnp.float32
    bundle = (jnp.dot(hlo_ref[...], wtlo_ref[...], preferred_element_type=f32, precision=_PREC)
              + jnp.dot(hhi_ref[...], wthi_ref[...], preferred_element_type=f32, precision=_PREC)
              + jnp.dot(clo_ref[...] * r, wblo_ref[...], preferred_element_type=f32, precision=_PREC)
              + jnp.dot(chi_ref[...] * r, wbhi_ref[...], preferred_element_type=f32, precision=_PREC)
              + b_ref[...])
    inv = 1.0 / jnp.maximum(
        jnp.sqrt(jnp.sum(bundle * bundle, axis=1, keepdims=True)), 1e-12)
    t = jnp.maximum(bundle * inv, 0.0) * bnsc_ref[...] + bnsh_ref[...]
    olo_ref[...] = hlo_ref[...] + t[:, :HH]
    ohi_ref[...] = hhi_ref[...] + t[:, HH:]


def _layer(h_lo, h_hi, c_lo, c_hi, deg16, Wt, Wb, b, bnsc, bnsh):
    R = 2000
    full = lambda i: (0, 0)
    blk = lambda i: (i, 0)
    return pl.pallas_call(
        _layer_body,
        grid=(N // R,),
        in_specs=[
            pl.BlockSpec((R, HH), blk), pl.BlockSpec((R, HH), blk),
            pl.BlockSpec((R, HH), blk), pl.BlockSpec((R, HH), blk),
            pl.BlockSpec((R, HH), blk),
            pl.BlockSpec((HH, H), full), pl.BlockSpec((HH, H), full),
            pl.BlockSpec((HH, H), full), pl.BlockSpec((HH, H), full),
            pl.BlockSpec((1, H), full), pl.BlockSpec((1, H), full),
            pl.BlockSpec((1, H), full),
        ],
        out_specs=(pl.BlockSpec((R, HH), blk), pl.BlockSpec((R, HH), blk)),
        out_shape=(jax.ShapeDtypeStruct((N, HH), jnp.float32),
                   jax.ShapeDtypeStruct((N, HH), jnp.float32)),
    )(h_lo, h_hi, c_lo, c_hi, deg16,
      Wt[:HH], Wt[HH:], Wb[:HH], Wb[HH:],
      b.reshape(1, H), bnsc.reshape(1, H), bnsh.reshape(1, H))


def _pack_bf16(v):
    vb = v.astype(jnp.bfloat16)
    lo = lax.bitcast_convert_type(vb[:, :HH], jnp.uint16).astype(jnp.uint32)
    hi = lax.bitcast_convert_type(vb[:, HH:], jnp.uint16).astype(jnp.uint32)
    return lax.bitcast_convert_type(lo | (hi << 16), jnp.float32)


def _unpack_bf16(w):
    u = lax.bitcast_convert_type(w, jnp.uint32)
    lo = lax.bitcast_convert_type((u & 0xFFFF).astype(jnp.uint16),
                                  jnp.bfloat16)
    hi = lax.bitcast_convert_type((u >> 16).astype(jnp.uint16),
                                  jnp.bfloat16)
    return lo, hi


def _ab_body(hlo_ref, hhi_ref, walo_ref, wahi_ref, wblo_ref, wbhi_ref,
             a_ref, b_ref):
    f32 = jnp.float32
    hlo = hlo_ref[...]
    hhi = hhi_ref[...]
    a = (jnp.dot(hlo, walo_ref[...], preferred_element_type=f32, precision=_PREC)
         + jnp.dot(hhi, wahi_ref[...], preferred_element_type=f32, precision=_PREC))
    b = (jnp.dot(hlo, wblo_ref[...], preferred_element_type=f32, precision=_PREC)
         + jnp.dot(hhi, wbhi_ref[...], preferred_element_type=f32, precision=_PREC))
    a_ref[...] = _pack_bf16(a)
    b_ref[...] = _pack_bf16(b)


def _ab(h_lo, h_hi, Wa, Wb):
    R = 2000
    full = lambda i: (0, 0)
    blk = lambda i: (i, 0)
    return pl.pallas_call(
        _ab_body,
        grid=(N // R,),
        in_specs=[
            pl.BlockSpec((R, HH), blk), pl.BlockSpec((R, HH), blk),
            pl.BlockSpec((HH, H), full), pl.BlockSpec((HH, H), full),
            pl.BlockSpec((HH, H), full), pl.BlockSpec((HH, H), full),
        ],
        out_specs=(pl.BlockSpec((R, HH), blk), pl.BlockSpec((R, HH), blk)),
        out_shape=(jax.ShapeDtypeStruct((N, HH), jnp.float32),
                   jax.ShapeDtypeStruct((N, HH), jnp.float32)),
    )(h_lo, h_hi, Wa[:HH], Wa[HH:], Wb[:HH], Wb[HH:])


def _mlp_body(sa_ref, sb_ref, b0lo_ref, b0hi_ref,
              w1lo_ref, w1hi_ref, b1_ref, w2_ref, b2_ref, o_ref):
    f32 = jnp.float32
    sa_lo, sa_hi = _unpack_bf16(sa_ref[...])
    sb_lo, sb_hi = _unpack_bf16(sb_ref[...])
    zero = jnp.bfloat16(0)
    y0_lo = jnp.maximum(sa_lo + sb_lo + b0lo_ref[...], zero)
    y0_hi = jnp.maximum(sa_hi + sb_hi + b0hi_ref[...], zero)
    y1 = jnp.maximum(
        jnp.dot(y0_lo, w1lo_ref[...], preferred_element_type=f32)
        + jnp.dot(y0_hi, w1hi_ref[...], preferred_element_type=f32)
        + b1_ref[...], 0.0)
    o_ref[...] = jnp.dot(y1, w2_ref[...], preferred_element_type=f32,
                         precision=_PREC) + b2_ref[...]


def _mlp(sa, sb, b0, W1, b1, W2, b2):
    R = 4000
    full = lambda i: (0, 0)
    blk = lambda i: (i, 0)
    bf = jnp.bfloat16
    return pl.pallas_call(
        _mlp_body,
        grid=(E // R,),
        in_specs=[
            pl.BlockSpec((R, HH), blk), pl.BlockSpec((R, HH), blk),
            pl.BlockSpec((1, HH), full), pl.BlockSpec((1, HH), full),
            pl.BlockSpec((HH, HH), full), pl.BlockSpec((HH, HH), full),
            pl.BlockSpec((1, HH), full),
            pl.BlockSpec((HH, 2), full), pl.BlockSpec((1, 2), full),
        ],
        out_specs=pl.BlockSpec((R, 2), blk),
        out_shape=jax.ShapeDtypeStruct((E, 2), jnp.float32),
    )(sa, sb, b0[:HH].reshape(1, HH).astype(bf), b0[HH:].reshape(1, HH).astype(bf),
      W1[:HH].astype(bf), W1[HH:].astype(bf), b1.reshape(1, HH),
      W2, b2.reshape(1, 2))


def kernel(x, edge_index, W_embed, b_embed,
           W_sage0, b_sage0, bn_g0, bn_b0, bn_m0, bn_v0,
           W_sage1, b_sage1, bn_g1, bn_b1, bn_m1, bn_v1,
           W_sage2, b_sage2, bn_g2, bn_b2, bn_m2, bn_v2,
           W_mlp0, b_mlp0, W_mlp1, b_mlp1, W_mlp2, b_mlp2):
    src = edge_index[0]
    dst = edge_index[1]
    srcp = jnp.pad(src.reshape(NSUB, ES),
                   ((0, 0), (0, PADE - ES))).reshape(NSUB, NCH, CW)
    dstp = jnp.pad(dst.reshape(NSUB, ES), ((0, 0), (0, PADE - ES)),
                   constant_values=N).reshape(NSUB, NCH, CW)
    zeros = jnp.zeros((CPR, HH), jnp.float32)
    ones_cw = jnp.ones((CW, HH), jnp.float32)

    deg16 = _deg_kernel(dstp, ones_cw, zeros)
    h_lo, h_hi = _embed(x, W_embed, b_embed)

    sages = ((W_sage0, b_sage0, bn_g0, bn_b0, bn_m0, bn_v0),
             (W_sage1, b_sage1, bn_g1, bn_b1, bn_m1, bn_v1),
             (W_sage2, b_sage2, bn_g2, bn_b2, bn_m2, bn_v2))
    for W, b, g, bb, m, v in sages:
        c_lo, c_hi = _seg_kernel(h_lo, h_hi, srcp, dstp, zeros)
        bnsc = g * jax.lax.rsqrt(v + BN_EPS)
        bnsh = bb - m * bnsc
        h_lo, h_hi = _layer(h_lo, h_hi, c_lo, c_hi, deg16,
                            W[:H], W[H:], b, bnsc, bnsh)

    a_t, b_t = _ab(h_lo, h_hi, W_mlp0[:H], W_mlp0[H:])
    sa, sb = _edge_gather_kernel(a_t, b_t, srcp, dstp)
    return _mlp(sa, sb, b_mlp0, W_mlp1, b_mlp1, W_mlp2, b_mlp2)


# speedup vs baseline: 4.9755x; 1.0014x over previous
import functools

import jax
import jax.numpy as jnp
from jax import lax
from jax.experimental import pallas as pl
from jax.experimental.pallas import tpu as pltpu
from jax.experimental.pallas import tpu_sc as plsc

N = 10000
E = 160000
DIN = 1024
H = 256
HH = 128
NSUB = 16
ES = E // NSUB
CW = 128
NCH = (ES + CW - 1) // CW
TAIL = ES - (NCH - 1) * CW
PADE = NCH * CW
NDUM = 8
NCOPY = 10
CPR = N // NCOPY
BN_EPS = 1e-5

_PREC = jax.lax.Precision.DEFAULT

_sc_mesh = plsc.VectorSubcoreMesh(core_axis_name="c", subcore_axis_name="s",
                                  num_cores=2, num_subcores=NSUB)


@functools.partial(
    pl.kernel,
    out_type=jax.ShapeDtypeStruct((N, HH), jnp.float32),
    mesh=_sc_mesh,
    scratch_types=[
        pltpu.VMEM((NCH, CW), jnp.int32),
        pltpu.VMEM((CW, HH), jnp.float32),
        pltpu.VMEM_SHARED((N + NDUM, HH), jnp.float32),
        pltpu.SemaphoreType.DMA,
    ],
)
def _deg_kernel(dstp, ones, zeros, deg_out, dst_v, ones_v, accum, sem):
    c = lax.axis_index("c")
    s = lax.axis_index("s")

    @pl.when(c == 0)
    def _():
        pl.when(s < NCOPY)(
            lambda: pltpu.sync_copy(zeros, accum.at[pl.ds(s * CPR, CPR)]))
        pltpu.sync_copy(dstp.at[s], dst_v)
        pltpu.sync_copy(ones, ones_v)
        plsc.subcore_barrier()

        for k in range(min(4, NCH)):
            pltpu.async_copy(ones_v, accum.at[dst_v.at[k]], sem, add=True)

        def chunk(j, carry):
            pltpu.make_async_copy(ones_v, accum.at[dst_v.at[j]], sem).wait()

            @pl.when(j + 4 < NCH)
            def _():
                pltpu.async_copy(ones_v, accum.at[dst_v.at[j + 4]], sem,
                                 add=True)
            return carry

        lax.fori_loop(0, NCH, chunk, 0)
        plsc.subcore_barrier()
        sl = pl.ds(s * CPR, CPR)
        pl.when(s < NCOPY)(
            lambda: pltpu.sync_copy(accum.at[sl], deg_out.at[sl]))


@functools.partial(
    pl.kernel,
    out_type=(jax.ShapeDtypeStruct((N, HH), jnp.float32),
              jax.ShapeDtypeStruct((N, HH), jnp.float32)),
    mesh=_sc_mesh,
    scratch_types=[
        pltpu.VMEM((4, CW), jnp.int32),
        pltpu.VMEM((4, CW), jnp.int32),
        pltpu.VMEM((3, CW, HH), jnp.float32),
        pltpu.VMEM_SHARED((N + NDUM, HH), jnp.float32),
        pltpu.SemaphoreType.DMA,
        pltpu.SemaphoreType.DMA,
    ],
)
def _seg_kernel(h_lo, h_hi, srcp, dstp, zeros, c_lo, c_hi,
                src_v, dst_v, rows_v, accum, sem, sem_i):
    c = lax.axis_index("c")
    s = lax.axis_index("s")
    pl.when(s < NCOPY)(
        lambda: pltpu.sync_copy(zeros, accum.at[pl.ds(s * CPR, CPR)]))
    pltpu.sync_copy(srcp.at[s, 0], src_v.at[0])
    pltpu.sync_copy(dstp.at[s, 0], dst_v.at[0])
    plsc.subcore_barrier()

    def idx_wait(k, slot):
        pltpu.make_async_copy(srcp.at[s, k], src_v.at[slot], sem_i).wait()
        pltpu.make_async_copy(dstp.at[s, k], dst_v.at[slot], sem_i).wait()

    def run(table):
        for k in (1, 2, 3):
            pltpu.async_copy(srcp.at[s, k], src_v.at[k], sem_i)
            pltpu.async_copy(dstp.at[s, k], dst_v.at[k], sem_i)
        pltpu.async_copy(table.at[src_v.at[0]], rows_v.at[0], sem)
        for k in (1, 2):
            idx_wait(k, k)
            pltpu.async_copy(table.at[src_v.at[k]], rows_v.at[k], sem)

        def chunk(j, carry):
            p3 = lax.rem(j, 3)
            p4 = lax.rem(j, 4)
            pltpu.make_async_copy(table.at[src_v.at[p4]],
                                  rows_v.at[p3], sem).wait()
            pltpu.sync_copy(rows_v.at[p3], accum.at[dst_v.at[p4]], add=True)

            @pl.when(j + 3 < NCH)
            def _():
                s4 = lax.rem(j + 3, 4)
                idx_wait(j + 3, s4)
                pltpu.async_copy(table.at[src_v.at[s4]], rows_v.at[p3], sem)

            @pl.when(j + 4 < NCH)
            def _():
                pltpu.async_copy(srcp.at[s, j + 4], src_v.at[p4], sem_i)
                pltpu.async_copy(dstp.at[s, j + 4], dst_v.at[p4], sem_i)
            return carry

        lax.fori_loop(0, NCH, chunk, 0)

    pl.when(c == 0)(lambda: run(h_lo))
    pl.when(c == 1)(lambda: run(h_hi))
    plsc.subcore_barrier()
    sl = pl.ds(s * CPR, CPR)

    @pl.when(s < NCOPY)
    def _():
        pl.when(c == 0)(lambda: pltpu.sync_copy(accum.at[sl], c_lo.at[sl]))
        pl.when(c == 1)(lambda: pltpu.sync_copy(accum.at[sl], c_hi.at[sl]))


@functools.partial(
    pl.kernel,
    out_type=(jax.ShapeDtypeStruct((E, HH), jnp.float32),
              jax.ShapeDtypeStruct((E, HH), jnp.float32)),
    mesh=_sc_mesh,
    scratch_types=[
        pltpu.VMEM((NCH, CW), jnp.int32),
        pltpu.VMEM((4, CW, HH), jnp.float32),
        pltpu.SemaphoreType.DMA,
    ],
)
def _edge_gather_kernel(a_t, b_t, srcp, dstp, sa, sb, idx_v, rows_v, sem):
    c = lax.axis_index("c")
    s = lax.axis_index("s")

    def run(table, idxs, out):
        pltpu.sync_copy(idxs.at[s], idx_v)
        for k in range(3):
            pltpu.async_copy(table.at[idx_v.at[k]], rows_v.at[k], sem)

        def chunk(j, carry):
            p = lax.rem(j, 4)
            pltpu.make_async_copy(table.at[idx_v.at[j]],
                                  rows_v.at[p], sem).wait()

            @pl.when(j + 3 < NCH)
            def _():
                pltpu.async_copy(table.at[idx_v.at[j + 3]],
                                 rows_v.at[lax.rem(j + 3, 4)], sem)
            pltpu.sync_copy(rows_v.at[p], out.at[pl.ds(s * ES + j * CW, CW)])
            return carry

        lax.fori_loop(0, NCH - 1, chunk, 0)
        p_last = (NCH - 1) % 4
        pltpu.make_async_copy(table.at[idx_v.at[NCH - 1]],
                              rows_v.at[p_last], sem).wait()
        pltpu.sync_copy(rows_v.at[p_last].at[pl.ds(0, TAIL)],
                        out.at[pl.ds(s * ES + (NCH - 1) * CW, TAIL)])

    pl.when(c == 0)(lambda: run(a_t, srcp, sa))
    pl.when(c == 1)(lambda: run(b_t, dstp, sb))


def _embed_body(x_ref, wlo_ref, whi_ref, blo_ref, bhi_ref, olo_ref, ohi_ref):
    x = x_ref[...]
    olo_ref[...] = jnp.dot(x, wlo_ref[...],
                           preferred_element_type=jnp.float32, precision=_PREC) + blo_ref[...]
    ohi_ref[...] = jnp.dot(x, whi_ref[...],
                           preferred_element_type=jnp.float32, precision=_PREC) + bhi_ref[...]


def _embed(x, W, b):
    R = 2000
    return pl.pallas_call(
        _embed_body,
        grid=(N // R,),
        in_specs=[
            pl.BlockSpec((R, DIN), lambda i: (i, 0)),
            pl.BlockSpec((DIN, HH), lambda i: (0, 0)),
            pl.BlockSpec((DIN, HH), lambda i: (0, 0)),
            pl.BlockSpec((1, HH), lambda i: (0, 0)),
            pl.BlockSpec((1, HH), lambda i: (0, 0)),
        ],
        out_specs=(pl.BlockSpec((R, HH), lambda i: (i, 0)),
                   pl.BlockSpec((R, HH), lambda i: (i, 0))),
        out_shape=(jax.ShapeDtypeStruct((N, HH), jnp.float32),
                   jax.ShapeDtypeStruct((N, HH), jnp.float32)),
    )(x, W[:, :HH], W[:, HH:], b[:HH].reshape(1, HH), b[HH:].reshape(1, HH))


def _layer_body(hlo_ref, hhi_ref, clo_ref, chi_ref, deg_ref,
                wtlo_ref, wthi_ref, wblo_ref, wbhi_ref,
                b_ref, bnsc_ref, bnsh_ref, olo_ref, ohi_ref):
    r = 1.0 / jnp.maximum(deg_ref[...][:, 0:1], 1.0)
    f32 = jnp.float32
    bundle = (jnp.dot(hlo_ref[...], wtlo_ref[...], preferred_element_type=f32, precision=_PREC)
              + jnp.dot(hhi_ref[...], wthi_ref[...], preferred_element_type=f32, precision=_PREC)
              + jnp.dot(clo_ref[...] * r, wblo_ref[...], preferred_element_type=f32, precision=_PREC)
              + jnp.dot(chi_ref[...] * r, wbhi_ref[...], preferred_element_type=f32, precision=_PREC)
              + b_ref[...])
    inv = 1.0 / jnp.maximum(
        jnp.sqrt(jnp.sum(bundle * bundle, axis=1, keepdims=True)), 1e-12)
    t = jnp.maximum(bundle * inv, 0.0) * bnsc_ref[...] + bnsh_ref[...]
    olo_ref[...] = hlo_ref[...] + t[:, :HH]
    ohi_ref[...] = hhi_ref[...] + t[:, HH:]


def _layer(h_lo, h_hi, c_lo, c_hi, deg16, Wt, Wb, b, bnsc, bnsh):
    R = 2000
    full = lambda i: (0, 0)
    blk = lambda i: (i, 0)
    return pl.pallas_call(
        _layer_body,
        grid=(N // R,),
        in_specs=[
            pl.BlockSpec((R, HH), blk), pl.BlockSpec((R, HH), blk),
            pl.BlockSpec((R, HH), blk), pl.BlockSpec((R, HH), blk),
            pl.BlockSpec((R, HH), blk),
            pl.BlockSpec((HH, H), full), pl.BlockSpec((HH, H), full),
            pl.BlockSpec((HH, H), full), pl.BlockSpec((HH, H), full),
            pl.BlockSpec((1, H), full), pl.BlockSpec((1, H), full),
            pl.BlockSpec((1, H), full),
        ],
        out_specs=(pl.BlockSpec((R, HH), blk), pl.BlockSpec((R, HH), blk)),
        out_shape=(jax.ShapeDtypeStruct((N, HH), jnp.float32),
                   jax.ShapeDtypeStruct((N, HH), jnp.float32)),
    )(h_lo, h_hi, c_lo, c_hi, deg16,
      Wt[:HH], Wt[HH:], Wb[:HH], Wb[HH:],
      b.reshape(1, H), bnsc.reshape(1, H), bnsh.reshape(1, H))


def _pack_bf16(v):
    vb = v.astype(jnp.bfloat16)
    lo = lax.bitcast_convert_type(vb[:, :HH], jnp.uint16).astype(jnp.uint32)
    hi = lax.bitcast_convert_type(vb[:, HH:], jnp.uint16).astype(jnp.uint32)
    return lax.bitcast_convert_type(lo | (hi << 16), jnp.float32)


def _unpack_bf16(w):
    u = lax.bitcast_convert_type(w, jnp.uint32)
    lo = lax.bitcast_convert_type((u & 0xFFFF).astype(jnp.uint16),
                                  jnp.bfloat16)
    hi = lax.bitcast_convert_type((u >> 16).astype(jnp.uint16),
                                  jnp.bfloat16)
    return lo, hi


def _ab_body(hlo_ref, hhi_ref, walo_ref, wahi_ref, wblo_ref, wbhi_ref,
             a_ref, b_ref):
    f32 = jnp.float32
    hlo = hlo_ref[...]
    hhi = hhi_ref[...]
    a = (jnp.dot(hlo, walo_ref[...], preferred_element_type=f32, precision=_PREC)
         + jnp.dot(hhi, wahi_ref[...], preferred_element_type=f32, precision=_PREC))
    b = (jnp.dot(hlo, wblo_ref[...], preferred_element_type=f32, precision=_PREC)
         + jnp.dot(hhi, wbhi_ref[...], preferred_element_type=f32, precision=_PREC))
    a_ref[...] = _pack_bf16(a)
    b_ref[...] = _pack_bf16(b)


def _ab(h_lo, h_hi, Wa, Wb):
    R = 2000
    full = lambda i: (0, 0)
    blk = lambda i: (i, 0)
    return pl.pallas_call(
        _ab_body,
        grid=(N // R,),
        in_specs=[
            pl.BlockSpec((R, HH), blk), pl.BlockSpec((R, HH), blk),
            pl.BlockSpec((HH, H), full), pl.BlockSpec((HH, H), full),
            pl.BlockSpec((HH, H), full), pl.BlockSpec((HH, H), full),
        ],
        out_specs=(pl.BlockSpec((R, HH), blk), pl.BlockSpec((R, HH), blk)),
        out_shape=(jax.ShapeDtypeStruct((N, HH), jnp.float32),
                   jax.ShapeDtypeStruct((N, HH), jnp.float32)),
    )(h_lo, h_hi, Wa[:HH], Wa[HH:], Wb[:HH], Wb[HH:])


def _mlp_body(sa_ref, sb_ref, b0lo_ref, b0hi_ref,
              w1lo_ref, w1hi_ref, b1_ref, w2_ref, b2_ref, o_ref):
    f32 = jnp.float32
    sa_lo, sa_hi = _unpack_bf16(sa_ref[...])
    sb_lo, sb_hi = _unpack_bf16(sb_ref[...])
    zero = jnp.bfloat16(0)
    y0_lo = jnp.maximum(sa_lo + sb_lo + b0lo_ref[...], zero)
    y0_hi = jnp.maximum(sa_hi + sb_hi + b0hi_ref[...], zero)
    y1 = jnp.maximum(
        jnp.dot(y0_lo, w1lo_ref[...], preferred_element_type=f32)
        + jnp.dot(y0_hi, w1hi_ref[...], preferred_element_type=f32)
        + b1_ref[...], 0.0)
    o_ref[...] = jnp.dot(y1, w2_ref[...], preferred_element_type=f32,
                         precision=_PREC) + b2_ref[...]


def _mlp(sa, sb, b0, W1, b1, W2, b2):
    R = 4000
    full = lambda i: (0, 0)
    blk = lambda i: (i, 0)
    bf = jnp.bfloat16
    return pl.pallas_call(
        _mlp_body,
        grid=(E // R,),
        in_specs=[
            pl.BlockSpec((R, HH), blk), pl.BlockSpec((R, HH), blk),
            pl.BlockSpec((1, HH), full), pl.BlockSpec((1, HH), full),
            pl.BlockSpec((HH, HH), full), pl.BlockSpec((HH, HH), full),
            pl.BlockSpec((1, HH), full),
            pl.BlockSpec((HH, 2), full), pl.BlockSpec((1, 2), full),
        ],
        out_specs=pl.BlockSpec((R, 2), blk),
        out_shape=jax.ShapeDtypeStruct((E, 2), jnp.float32),
    )(sa, sb, b0[:HH].reshape(1, HH).astype(bf), b0[HH:].reshape(1, HH).astype(bf),
      W1[:HH].astype(bf), W1[HH:].astype(bf), b1.reshape(1, HH),
      W2, b2.reshape(1, 2))


def kernel(x, edge_index, W_embed, b_embed,
           W_sage0, b_sage0, bn_g0, bn_b0, bn_m0, bn_v0,
           W_sage1, b_sage1, bn_g1, bn_b1, bn_m1, bn_v1,
           W_sage2, b_sage2, bn_g2, bn_b2, bn_m2, bn_v2,
           W_mlp0, b_mlp0, W_mlp1, b_mlp1, W_mlp2, b_mlp2):
    src = edge_index[0]
    dst = edge_index[1]
    srcp = jnp.pad(src.reshape(NSUB, ES),
                   ((0, 0), (0, PADE - ES))).reshape(NSUB, NCH, CW)
    dstp = jnp.pad(dst.reshape(NSUB, ES), ((0, 0), (0, PADE - ES)),
                   constant_values=N).reshape(NSUB, NCH, CW)
    zeros = jnp.zeros((CPR, HH), jnp.float32)
    ones_cw = jnp.ones((CW, HH), jnp.float32)

    deg16 = _deg_kernel(dstp, ones_cw, zeros)
    h_lo, h_hi = _embed(x, W_embed, b_embed)

    sages = ((W_sage0, b_sage0, bn_g0, bn_b0, bn_m0, bn_v0),
             (W_sage1, b_sage1, bn_g1, bn_b1, bn_m1, bn_v1),
             (W_sage2, b_sage2, bn_g2, bn_b2, bn_m2, bn_v2))
    for W, b, g, bb, m, v in sages:
        c_lo, c_hi = _seg_kernel(h_lo, h_hi, srcp, dstp, zeros)
        bnsc = g * jax.lax.rsqrt(v + BN_EPS)
        bnsh = bb - m * bnsc
        h_lo, h_hi = _layer(h_lo, h_hi, c_lo, c_hi, deg16,
                            W[:H], W[H:], b, bnsc, bnsh)

    a_t, b_t = _ab(h_lo, h_hi, W_mlp0[:H], W_mlp0[H:])
    sa, sb = _edge_gather_kernel(a_t, b_t, srcp, dstp)
    return _mlp(sa, sb, b_mlp0, W_mlp1, b_mlp1, W_mlp2, b_mlp2)
